# Initial kernel scaffold; baseline (speedup 1.0000x reference)
import functools

import jax
import jax.numpy as jnp
from jax import lax
from jax.experimental import pallas as pl
from jax.experimental.pallas import tpu as pltpu
from jax.experimental.pallas import tpu_sc as plsc

N = 10000
E = 320000
NP = 10240
HEADS = 2
F = 128
FH = 64
CHUNK = 400
NCORE = 2
NSUB = 16
NWORK = NCORE * NSUB

_mesh = lambda: plsc.VectorSubcoreMesh(
    core_axis_name="c", subcore_axis_name="s", num_cores=NCORE,
    num_subcores=NSUB)

_f32 = jnp.float32
_i32 = jnp.int32


def _iota16():
    return lax.iota(_i32, 16)


def _sc_degrees(ei, z1, ones):
    kfn = pl.kernel(
        _degrees_body,
        out_type=jax.ShapeDtypeStruct((4 * NP,), _f32),
        mesh=_mesh(),
        scratch_types=[
            pltpu.VMEM_SHARED((NP,), _f32),
            pltpu.VMEM_SHARED((NP,), _f32),
            pltpu.VMEM((E // NWORK,), _i32),
            pltpu.VMEM((E // NWORK,), _i32),
            pltpu.VMEM((E // NWORK,), _f32),
        ],
    )
    return kfn(ei, z1, ones)


def _degrees_body(ei, z1, ones, out, dego, degi, srcv, dstv, onesv):
    c = lax.axis_index("c")
    s = lax.axis_index("s")
    wid = c * NSUB + s
    per = E // NWORK
    base = wid * per
    seg = NP // NSUB
    pltpu.sync_copy(z1.at[0, pl.ds(s * seg, seg)], dego.at[pl.ds(s * seg, seg)])
    pltpu.sync_copy(z1.at[1, pl.ds(s * seg, seg)], degi.at[pl.ds(s * seg, seg)])
    pltpu.sync_copy(ei.at[0, pl.ds(base, per)], srcv)
    pltpu.sync_copy(ei.at[1, pl.ds(base, per)], dstv)
    pltpu.sync_copy(ones, onesv)
    plsc.subcore_barrier()
    pltpu.sync_copy(onesv, dego.at[srcv], add=True)
    pltpu.sync_copy(onesv, degi.at[dstv], add=True)
    plsc.subcore_barrier()
    pltpu.sync_copy(dego.at[pl.ds(s * seg, seg)],
                    out.at[pl.ds((c * 2 + 0) * NP + s * seg, seg)])
    pltpu.sync_copy(degi.at[pl.ds(s * seg, seg)],
                    out.at[pl.ds((c * 2 + 1) * NP + s * seg, seg)])


def _sc_conv1(ei, tf, z2):
    kfn = pl.kernel(
        _conv1_body,
        out_type=jax.ShapeDtypeStruct((2 * NP, FH), _f32),
        mesh=_mesh(),
        scratch_types=[
            pltpu.VMEM_SHARED((NP, FH), _f32),
            pltpu.VMEM((CHUNK,), _i32),
            pltpu.VMEM((CHUNK,), _i32),
            pltpu.VMEM((CHUNK, FH), _f32),
            pltpu.SemaphoreType.DMA,
        ],
    )
    return kfn(ei, tf, z2)


def _conv1_body(ei, tf, z2, out, acc, srcv, dstv, rows, sem):
    c = lax.axis_index("c")
    s = lax.axis_index("s")
    seg = NP // NSUB
    pltpu.sync_copy(z2.at[pl.ds(s * seg, seg)], acc.at[pl.ds(s * seg, seg)])
    plsc.subcore_barrier()
    per = E // NSUB
    nchunk = per // CHUNK
    delta = c * N

    def chunk(i, _):
        base = s * per + i * CHUNK
        pltpu.sync_copy(ei.at[0, pl.ds(base, CHUNK)], srcv)
        pltpu.sync_copy(ei.at[1, pl.ds(base, CHUNK)], dstv)
        for g in range(CHUNK // 16):
            srcv[pl.ds(g * 16, 16)] = srcv[pl.ds(g * 16, 16)] + delta
        pltpu.async_copy(tf.at[srcv], rows, sem).wait()
        pltpu.sync_copy(rows, acc.at[dstv], add=True)
        return _

    lax.fori_loop(0, nchunk, chunk, 0)
    plsc.subcore_barrier()
    pltpu.sync_copy(acc.at[pl.ds(s * seg, seg)],
                    out.at[pl.ds(c * NP + s * seg, seg)])


def _sc_edgescalars(ei, att, zc, z1):
    kfn = pl.kernel(
        _edgescalars_body,
        out_type=(
            jax.ShapeDtypeStruct((HEADS, E), _i32),
            jax.ShapeDtypeStruct((HEADS, E), _i32),
            jax.ShapeDtypeStruct((4 * 2 * NP,), _f32),
            jax.ShapeDtypeStruct((2 * NP,), _f32),
        ),
        mesh=_mesh(),
        scratch_types=[
            pltpu.VMEM_SHARED((2 * NP,), _f32),
            pltpu.VMEM_SHARED((2 * NP,), _f32),
            pltpu.VMEM_SHARED((NP,), _f32),
            pltpu.VMEM((CHUNK,), _i32),
            pltpu.VMEM((CHUNK,), _i32),
            pltpu.VMEM((CHUNK, 8), _f32),
            pltpu.VMEM((CHUNK, 8), _f32),
            pltpu.VMEM((CHUNK,), _i32),
            pltpu.VMEM((CHUNK,), _i32),
            pltpu.VMEM((CHUNK,), _i32),
            pltpu.VMEM((CHUNK,), _i32),
            pltpu.VMEM((CHUNK,), _f32),
            pltpu.VMEM((CHUNK,), _f32),
            pltpu.VMEM((CHUNK,), _f32),
            pltpu.SemaphoreType.DMA,
        ],
    )
    return kfn(ei, att, zc, z1)


def _edgescalars_body(ei, att, zc, z1, srcp_o, dstp_o, eap_o, cap_o,
                      eacc0, eacc1, cacc, srcv, dstv, sa, da,
                      spb0, spb1, dpb0, dpb1, valb0, valb1, ndb, sem):
    c = lax.axis_index("c")
    s = lax.axis_index("s")
    wid = c * NSUB + s
    seg2 = 2 * NP // NSUB
    seg = NP // NSUB
    pltpu.sync_copy(zc.at[0, pl.ds(s * seg2, seg2)], eacc0.at[pl.ds(s * seg2, seg2)])
    pltpu.sync_copy(zc.at[1, pl.ds(s * seg2, seg2)], eacc1.at[pl.ds(s * seg2, seg2)])
    pltpu.sync_copy(z1.at[0, pl.ds(s * seg, seg)], cacc.at[pl.ds(s * seg, seg)])
    plsc.subcore_barrier()
    per = E // NWORK
    nchunk = per // CHUNK

    def chunk(i, _):
        base = wid * per + i * CHUNK
        pltpu.sync_copy(ei.at[0, pl.ds(base, CHUNK)], srcv)
        pltpu.sync_copy(ei.at[1, pl.ds(base, CHUNK)], dstv)
        d1 = pltpu.async_copy(att.at[srcv], sa, sem)
        d2 = pltpu.async_copy(att.at[dstv], da, sem)
        d1.wait()
        d2.wait()
        for g in range(CHUNK // 16):
            sl = pl.ds(g * 16, 16)
            rows = g * 16 + _iota16()
            srcs = srcv[sl]
            dsts = dstv[sl]
            nd16 = plsc.load_gather(da, [rows, jnp.full((16,), 4, _i32)])
            ndb[sl] = nd16
            for h, spb, dpb, valb in ((0, spb0, dpb0, valb0),
                                      (1, spb1, dpb1, valb1)):
                el = plsc.load_gather(sa, [rows, jnp.full((16,), h, _i32)])
                er = plsc.load_gather(da, [rows, jnp.full((16,), 2 + h, _i32)])
                t = el + er
                neg = t <= 0.0
                bi = jnp.where(neg, _i32(1), _i32(0))
                spb[sl] = srcs + N * bi
                dpb[sl] = dsts + NP * bi
                valb[sl] = jnp.exp(el * jnp.where(neg, _f32(0.2), _f32(1.0)))
        pltpu.sync_copy(spb0, srcp_o.at[0, pl.ds(base, CHUNK)])
        pltpu.sync_copy(spb1, srcp_o.at[1, pl.ds(base, CHUNK)])
        pltpu.sync_copy(dpb0, dstp_o.at[0, pl.ds(base, CHUNK)])
        pltpu.sync_copy(dpb1, dstp_o.at[1, pl.ds(base, CHUNK)])
        pltpu.sync_copy(valb0, eacc0.at[dpb0], add=True)
        pltpu.sync_copy(valb1, eacc1.at[dpb1], add=True)
        pltpu.sync_copy(ndb, cacc.at[srcv], add=True)
        return _

    lax.fori_loop(0, nchunk, chunk, 0)
    plsc.subcore_barrier()
    pltpu.sync_copy(eacc0.at[pl.ds(s * seg2, seg2)],
                    eap_o.at[pl.ds((c * 2 + 0) * 2 * NP + s * seg2, seg2)])
    pltpu.sync_copy(eacc1.at[pl.ds(s * seg2, seg2)],
                    eap_o.at[pl.ds((c * 2 + 1) * 2 * NP + s * seg2, seg2)])
    pltpu.sync_copy(cacc.at[pl.ds(s * seg, seg)],
                    cap_o.at[pl.ds(c * NP + s * seg, seg)])


def _sc_gat(srcp, dstp, gf, z2):
    kfn = pl.kernel(
        _gat_body,
        out_type=jax.ShapeDtypeStruct((4 * 2 * NP, FH), _f32),
        mesh=_mesh(),
        scratch_types=[
            pltpu.VMEM_SHARED((2 * NP, FH), _f32),
            pltpu.VMEM((CHUNK,), _i32),
            pltpu.VMEM((CHUNK,), _i32),
            pltpu.VMEM((CHUNK, FH), _f32),
            pltpu.SemaphoreType.DMA,
        ],
    )
    return kfn(srcp, dstp, gf, z2)


def _gat_body(srcp, dstp, gf, z2, out, acc, srcv, dstv, rows, sem):
    c = lax.axis_index("c")
    s = lax.axis_index("s")
    seg2 = 2 * NP // NSUB
    per = E // NSUB
    nchunk = per // CHUNK
    for p in range(2):
        pltpu.sync_copy(z2.at[pl.ds(s * seg2, seg2)], acc.at[pl.ds(s * seg2, seg2)])
        plsc.subcore_barrier()
        delta = (c * 2 + p) * (2 * N)

        def chunk(i, _):
            base = s * per + i * CHUNK
            pltpu.sync_copy(srcp.at[c, pl.ds(base, CHUNK)], srcv)
            pltpu.sync_copy(dstp.at[c, pl.ds(base, CHUNK)], dstv)
            for g in range(CHUNK // 16):
                srcv[pl.ds(g * 16, 16)] = srcv[pl.ds(g * 16, 16)] + delta
            pltpu.async_copy(gf.at[srcv], rows, sem).wait()
            pltpu.sync_copy(rows, acc.at[dstv], add=True)
            return _

        lax.fori_loop(0, nchunk, chunk, 0)
        plsc.subcore_barrier()
        pltpu.sync_copy(
            acc.at[pl.ds(s * seg2, seg2)],
            out.at[pl.ds((c * 2 + p) * 2 * NP + s * seg2, seg2)])
        plsc.subcore_barrier()


_BLK = 1000


def _tc_matmul_xw1(x, w1):
    def body(x_ref, w_ref, o_ref):
        o_ref[...] = jnp.dot(x_ref[...], w_ref[...],
                             preferred_element_type=_f32)

    return pl.pallas_call(
        body,
        grid=(N // _BLK,),
        in_specs=[
            pl.BlockSpec((_BLK, F), lambda i: (i, 0)),
            pl.BlockSpec((F, F), lambda i: (0, 0)),
        ],
        out_specs=pl.BlockSpec((_BLK, F), lambda i: (i, 0)),
        out_shape=jax.ShapeDtypeStruct((N, F), _f32),
    )(x, w1)


def _tc_norms(t1, d00, d01, d10, d11):
    def body(t1_ref, a_ref, b_ref, cc_ref, d_ref, t1n_ref, ns_ref, nd_ref):
        dego = a_ref[0] + cc_ref[0]
        degi = b_ref[0] + d_ref[0]
        ns = lax.rsqrt(jnp.maximum(dego, 1.0))
        nd = lax.rsqrt(jnp.maximum(degi, 1.0))
        t1n_ref[...] = t1_ref[...] * ns[:, None]
        ns_ref[0] = ns
        nd_ref[0] = nd

    vec = pl.BlockSpec((1, _BLK), lambda i: (i, 0))
    return pl.pallas_call(
        body,
        grid=(N // _BLK,),
        in_specs=[pl.BlockSpec((_BLK, F), lambda i: (i, 0)), vec, vec, vec, vec],
        out_specs=[pl.BlockSpec((_BLK, F), lambda i: (i, 0)), vec, vec],
        out_shape=[
            jax.ShapeDtypeStruct((N, F), _f32),
            jax.ShapeDtypeStruct((N // _BLK, _BLK), _f32),
            jax.ShapeDtypeStruct((N // _BLK, _BLK), _f32),
        ],
    )(t1, d00, d01, d10, d11)


def _tc_gatprep(m0, m1, nd, b1, wg, al, ar):
    def body(m0_ref, m1_ref, nd_ref, b1_ref, wg_ref, al_ref, ar_ref,
             att_ref, ebd_ref, g1_ref, g2_ref):
        nd = nd_ref[0]
        m = jnp.concatenate([m0_ref[...], m1_ref[...]], axis=1)
        h = m * nd[:, None] + b1_ref[0]
        nrm = jnp.sqrt(jnp.sum(h * h, axis=1, keepdims=True))
        h = h / jnp.maximum(nrm, 1e-12)
        h = jax.nn.sigmoid(h)
        feat = jnp.dot(h, wg_ref[...], preferred_element_type=_f32)
        cols = []
        for hh in range(HEADS):
            f = feat[:, hh * F:(hh + 1) * F]
            el = jnp.sum(f * al_ref[hh], axis=1)
            er = jnp.sum(f * ar_ref[hh], axis=1)
            g1_ref[hh] = f * jnp.exp(el)[:, None]
            g2_ref[hh] = f * jnp.exp(0.2 * el)[:, None]
            cols.append((el, er))
        (el0, er0), (el1, er1) = cols
        att_ref[...] = jnp.stack(
            [el0, el1, er0, er1, nd, jnp.zeros_like(nd), jnp.zeros_like(nd),
             jnp.zeros_like(nd)], axis=1)
        ebd_ref[...] = jnp.stack(
            [jnp.exp(er0), jnp.exp(er1), jnp.exp(0.2 * er0),
             jnp.exp(0.2 * er1)], axis=1)

    vec = pl.BlockSpec((1, _BLK), lambda i: (i, 0))
    half = pl.BlockSpec((_BLK, FH), lambda i: (i, 0))
    return pl.pallas_call(
        body,
        grid=(N // _BLK,),
        in_specs=[
            half, half, vec,
            pl.BlockSpec((1, F), lambda i: (0, 0)),
            pl.BlockSpec((F, HEADS * F), lambda i: (0, 0)),
            pl.BlockSpec((HEADS, F), lambda i: (0, 0)),
            pl.BlockSpec((HEADS, F), lambda i: (0, 0)),
        ],
        out_specs=[
            pl.BlockSpec((_BLK, 8), lambda i: (i, 0)),
            pl.BlockSpec((_BLK, 4), lambda i: (i, 0)),
            pl.BlockSpec((HEADS, _BLK, F), lambda i: (0, i, 0)),
            pl.BlockSpec((HEADS, _BLK, F), lambda i: (0, i, 0)),
        ],
        out_shape=[
            jax.ShapeDtypeStruct((N, 8), _f32),
            jax.ShapeDtypeStruct((N, 4), _f32),
            jax.ShapeDtypeStruct((HEADS, N, F), _f32),
            jax.ShapeDtypeStruct((HEADS, N, F), _f32),
        ],
    )(m0, m1, nd, b1, wg, al, ar)


_RBLK = 80


def _tc_readout(num, eap, cap, ebd, ns, bg, w2, b2, wc, bc):
    nblk = N // _RBLK
    negoff = NP // _RBLK

    def body(np_ref, nn_ref, eapp_ref, eapn_ref, cap_ref, ebd_ref, ns_ref,
             bg_ref, w2_ref, b2_ref, wc_ref, bc_ref, o_ref, wacc):
        i = pl.program_id(0)

        @pl.when(i == 0)
        def _():
            wacc[...] = jnp.zeros_like(wacc)

        cc = cap_ref[0] + cap_ref[1]
        sw = ns_ref[0] * cc
        ws = []
        for hh in range(HEADS):
            eb = ebd_ref[:, hh]
            ed = ebd_ref[:, 2 + hh]
            pos = jnp.concatenate([np_ref[hh, 0], np_ref[hh, 1]], axis=1)
            ngt = jnp.concatenate([nn_ref[hh, 0], nn_ref[hh, 1]], axis=1)
            num_h = eb[:, None] * pos + ed[:, None] * ngt
            esum = (eb * (eapp_ref[0, hh] + eapp_ref[1, hh])
                    + ed * (eapn_ref[0, hh] + eapn_ref[1, hh]))
            h2 = jax.nn.relu(num_h / (esum[:, None] + 1e-9) + bg_ref[hh])
            ws.append(jnp.dot(sw[None, :], h2, preferred_element_type=_f32))
        wacc[...] += jnp.concatenate(ws, axis=0)

        @pl.when(i == nblk - 1)
        def _():
            hg = jnp.dot(wacc[...] * (1.0 / N), w2_ref[...],
                         preferred_element_type=_f32) + b2_ref[0]
            o_ref[...] = jnp.dot(hg, wc_ref[...],
                                 preferred_element_type=_f32) + bc_ref[0]

    full = lambda shape: pl.BlockSpec(shape, lambda i: tuple(0 for _ in shape))
    return pl.pallas_call(
        body,
        grid=(nblk,),
        in_specs=[
            pl.BlockSpec((HEADS, 2, _RBLK, FH), lambda i: (0, 0, i, 0)),
            pl.BlockSpec((HEADS, 2, _RBLK, FH), lambda i: (0, 0, negoff + i, 0)),
            pl.BlockSpec((2, HEADS, _RBLK), lambda i: (0, 0, i)),
            pl.BlockSpec((2, HEADS, _RBLK), lambda i: (0, 0, negoff + i)),
            pl.BlockSpec((2, _RBLK), lambda i: (0, i)),
            pl.BlockSpec((_RBLK, 4), lambda i: (i, 0)),
            pl.BlockSpec((1, _RBLK), lambda i: (i, 0)),
            full((HEADS, F)),
            full((F, F)),
            full((1, F)),
            full((F, 16)),
            full((1, 16)),
        ],
        out_specs=full((HEADS, 16)),
        out_shape=jax.ShapeDtypeStruct((HEADS, 16), _f32),
        scratch_shapes=[pltpu.VMEM((HEADS, F), _f32)],
    )(num, num, eap, eap, cap, ebd, ns, bg, w2, b2, wc, bc)


def kernel(x, edge_index, W1, b1, Wg, al, ar, bg, W2, b2, Wc, bc):
    ei = edge_index
    z1 = jnp.zeros((2, NP), _f32)
    z2 = jnp.zeros((2 * NP, FH), _f32)
    zc = jnp.zeros((2, 2 * NP), _f32)
    ones = jnp.ones((E // NWORK,), _f32)

    deg = _sc_degrees(ei, z1, ones)
    t1 = _tc_matmul_xw1(x, W1)

    dr = lambda k: deg[k * NP:k * NP + N].reshape(N // _BLK, _BLK)
    t1n, ns, nd = _tc_norms(t1, dr(0), dr(2), dr(1), dr(3))

    tf = jnp.stack([t1n[:, :FH], t1n[:, FH:]]).reshape(2 * N, FH)
    mflat = _sc_conv1(ei, tf, z2)
    m0 = mflat[:N]
    m1 = mflat[NP:NP + N]

    att, ebd, g1, g2 = _tc_gatprep(m0, m1, nd, b1.reshape(1, F), Wg, al, ar)

    srcp, dstp, eap, cap = _sc_edgescalars(ei, att, zc, z1)

    gcat = jnp.concatenate([g1, g2], axis=1)
    gf = jnp.stack([gcat[0, :, :FH], gcat[0, :, FH:],
                    gcat[1, :, :FH], gcat[1, :, FH:]]).reshape(4 * 2 * N, FH)
    numflat = _sc_gat(srcp, dstp, gf, z2)

    out = _tc_readout(
        numflat.reshape(HEADS, 2, 2 * NP, FH),
        eap.reshape(2, HEADS, 2 * NP),
        cap.reshape(2, NP),
        ebd,
        ns.reshape(N // _RBLK, _RBLK),
        bg, W2, b2.reshape(1, F), Wc, bc.reshape(1, 16))
    return out.reshape(1, HEADS, 16)


# speedup vs baseline: 50.5931x; 50.5931x over previous
import jax
import jax.numpy as jnp
from jax import lax
from jax.experimental import pallas as pl
from jax.experimental.pallas import tpu as pltpu
from jax.experimental.pallas import tpu_sc as plsc

N = 10000
E = 320000
NP = 10240
HEADS = 2
F = 128
CHUNK = 400
CHUNKR = 80
NCORE = 2
NSUB = 16
NWORK = NCORE * NSUB

_mesh = lambda: plsc.VectorSubcoreMesh(
    core_axis_name="c", subcore_axis_name="s", num_cores=NCORE,
    num_subcores=NSUB)

_f32 = jnp.float32
_i32 = jnp.int32


def _iota16():
    return lax.iota(_i32, 16)


def _sc_degrees(ei, z1, ones):
    kfn = pl.kernel(
        _degrees_body,
        out_type=jax.ShapeDtypeStruct((4 * NP,), _f32),
        mesh=_mesh(),
        scratch_types=[
            pltpu.VMEM_SHARED((NP,), _f32),
            pltpu.VMEM_SHARED((NP,), _f32),
            pltpu.VMEM((E // NWORK,), _i32),
            pltpu.VMEM((E // NWORK,), _i32),
            pltpu.VMEM((E // NWORK,), _f32),
        ],
    )
    return kfn(ei, z1, ones)


def _degrees_body(ei, z1, ones, out, dego, degi, srcv, dstv, onesv):
    c = lax.axis_index("c")
    s = lax.axis_index("s")
    wid = c * NSUB + s
    per = E // NWORK
    base = wid * per
    seg = NP // NSUB
    pltpu.sync_copy(z1.at[pl.ds(s * seg, seg)], dego.at[pl.ds(s * seg, seg)])
    pltpu.sync_copy(z1.at[pl.ds(s * seg, seg)], degi.at[pl.ds(s * seg, seg)])
    pltpu.sync_copy(ei.at[pl.ds(base, per)], srcv)
    pltpu.sync_copy(ei.at[pl.ds(E + base, per)], dstv)
    pltpu.sync_copy(ones, onesv)
    plsc.subcore_barrier()
    pltpu.sync_copy(onesv, dego.at[srcv], add=True)
    pltpu.sync_copy(onesv, degi.at[dstv], add=True)
    plsc.subcore_barrier()
    pltpu.sync_copy(dego.at[pl.ds(s * seg, seg)],
                    out.at[pl.ds((c * 2 + 0) * NP + s * seg, seg)])
    pltpu.sync_copy(degi.at[pl.ds(s * seg, seg)],
                    out.at[pl.ds((c * 2 + 1) * NP + s * seg, seg)])


def _sc_conv1(ei, t1n, z2):
    kfn = pl.kernel(
        _conv1_body,
        out_type=jax.ShapeDtypeStruct((2 * NP, F), _f32),
        mesh=_mesh(),
        scratch_types=[
            pltpu.VMEM_SHARED((NP, F), _f32),
            pltpu.VMEM((CHUNKR,), _i32),
            pltpu.VMEM((CHUNKR,), _i32),
            pltpu.VMEM((CHUNKR, F), _f32),
            pltpu.SemaphoreType.DMA,
        ],
    )
    return kfn(ei, t1n, z2)


def _conv1_body(ei, t1n, z2, out, acc, srcv, dstv, rows, sem):
    c = lax.axis_index("c")
    s = lax.axis_index("s")
    seg = NP // NSUB
    pltpu.sync_copy(z2.at[pl.ds(s * seg, seg)], acc.at[pl.ds(s * seg, seg)])
    plsc.subcore_barrier()
    per = E // NWORK
    nchunk = per // CHUNKR
    wid = c * NSUB + s

    def chunk(i, _):
        base = wid * per + i * CHUNKR
        pltpu.sync_copy(ei.at[pl.ds(base, CHUNKR)], srcv)
        pltpu.sync_copy(ei.at[pl.ds(E + base, CHUNKR)], dstv)
        pltpu.async_copy(t1n.at[srcv], rows, sem).wait()
        pltpu.sync_copy(rows, acc.at[dstv], add=True)
        return _

    lax.fori_loop(0, nchunk, chunk, 0)
    plsc.subcore_barrier()
    pltpu.sync_copy(acc.at[pl.ds(s * seg, seg)],
                    out.at[pl.ds(c * NP + s * seg, seg)])


def _sc_edgescalars(ei, el0, el1, er0, er1, ndt, z1):
    kfn = pl.kernel(
        _edgescalars_body,
        out_type=(
            jax.ShapeDtypeStruct((HEADS * E,), _f32),
            jax.ShapeDtypeStruct((4 * NP,), _f32),
            jax.ShapeDtypeStruct((2 * NP,), _f32),
        ),
        mesh=_mesh(),
        scratch_types=[
            pltpu.VMEM_SHARED((NP,), _f32),
            pltpu.VMEM_SHARED((NP,), _f32),
            pltpu.VMEM_SHARED((NP,), _f32),
            pltpu.VMEM((CHUNK,), _i32),
            pltpu.VMEM((CHUNK,), _i32),
            pltpu.VMEM((CHUNK,), _f32),
            pltpu.VMEM((CHUNK,), _f32),
            pltpu.VMEM((CHUNK,), _f32),
            pltpu.VMEM((CHUNK,), _f32),
            pltpu.VMEM((CHUNK,), _f32),
            pltpu.VMEM((CHUNK,), _f32),
            pltpu.VMEM((CHUNK,), _f32),
            pltpu.SemaphoreType.DMA,
        ],
    )
    return kfn(ei, el0, el1, er0, er1, ndt, z1)


def _edgescalars_body(ei, el0, el1, er0, er1, ndt, z1, ex_o, eap_o, cap_o,
                      eacc0, eacc1, cacc, srcv, dstv,
                      el0b, el1b, er0b, er1b, ndb, exb0, exb1, sem):
    c = lax.axis_index("c")
    s = lax.axis_index("s")
    wid = c * NSUB + s
    seg = NP // NSUB
    pltpu.sync_copy(z1.at[pl.ds(s * seg, seg)], eacc0.at[pl.ds(s * seg, seg)])
    pltpu.sync_copy(z1.at[pl.ds(s * seg, seg)], eacc1.at[pl.ds(s * seg, seg)])
    pltpu.sync_copy(z1.at[pl.ds(s * seg, seg)], cacc.at[pl.ds(s * seg, seg)])
    plsc.subcore_barrier()
    per = E // NWORK
    nchunk = per // CHUNK

    def chunk(i, _):
        base = wid * per + i * CHUNK
        pltpu.sync_copy(ei.at[pl.ds(base, CHUNK)], srcv)
        pltpu.sync_copy(ei.at[pl.ds(E + base, CHUNK)], dstv)
        ds = [pltpu.async_copy(el0.at[srcv], el0b, sem),
              pltpu.async_copy(el1.at[srcv], el1b, sem),
              pltpu.async_copy(er0.at[dstv], er0b, sem),
              pltpu.async_copy(er1.at[dstv], er1b, sem),
              pltpu.async_copy(ndt.at[dstv], ndb, sem)]
        for d in ds:
            d.wait()
        for g in range(CHUNK // 16):
            sl = pl.ds(g * 16, 16)
            for elb, erb, exb in ((el0b, er0b, exb0), (el1b, er1b, exb1)):
                t = elb[sl] + erb[sl]
                lr = jnp.where(t > 0.0, t, 0.2 * t)
                exb[sl] = jnp.exp(lr)
        pltpu.sync_copy(exb0, ex_o.at[pl.ds(base, CHUNK)])
        pltpu.sync_copy(exb1, ex_o.at[pl.ds(E + base, CHUNK)])
        pltpu.sync_copy(exb0, eacc0.at[dstv], add=True)
        pltpu.sync_copy(exb1, eacc1.at[dstv], add=True)
        pltpu.sync_copy(ndb, cacc.at[srcv], add=True)
        return _

    lax.fori_loop(0, nchunk, chunk, 0)
    plsc.subcore_barrier()
    pltpu.sync_copy(eacc0.at[pl.ds(s * seg, seg)],
                    eap_o.at[pl.ds((c * 2 + 0) * NP + s * seg, seg)])
    pltpu.sync_copy(eacc1.at[pl.ds(s * seg, seg)],
                    eap_o.at[pl.ds((c * 2 + 1) * NP + s * seg, seg)])
    pltpu.sync_copy(cacc.at[pl.ds(s * seg, seg)],
                    cap_o.at[pl.ds(c * NP + s * seg, seg)])


def _sc_gat(ei, exf, ft, z2):
    kfn = pl.kernel(
        _gat_body,
        out_type=jax.ShapeDtypeStruct((2 * NP, F), _f32),
        mesh=_mesh(),
        scratch_types=[
            pltpu.VMEM_SHARED((NP, F), _f32),
            pltpu.VMEM((CHUNKR,), _i32),
            pltpu.VMEM((CHUNKR,), _i32),
            pltpu.VMEM((CHUNKR,), _f32),
            pltpu.VMEM((CHUNKR, F), _f32),
            pltpu.SemaphoreType.DMA,
        ],
    )
    return kfn(ei, exf, ft, z2)


def _gat_body(ei, exf, ft, z2, out, acc, srcv, dstv, exv, rows, sem):
    c = lax.axis_index("c")
    s = lax.axis_index("s")
    seg = NP // NSUB
    pltpu.sync_copy(z2.at[pl.ds(s * seg, seg)], acc.at[pl.ds(s * seg, seg)])
    plsc.subcore_barrier()
    per = E // NSUB
    nchunk = per // CHUNKR
    delta = c * N

    def chunk(i, _):
        base = s * per + i * CHUNKR
        pltpu.sync_copy(ei.at[pl.ds(base, CHUNKR)], srcv)
        pltpu.sync_copy(ei.at[pl.ds(E + base, CHUNKR)], dstv)
        pltpu.sync_copy(exf.at[pl.ds(c * E + base, CHUNKR)], exv)
        for g in range(CHUNKR // 16):
            srcv[pl.ds(g * 16, 16)] = srcv[pl.ds(g * 16, 16)] + delta
        pltpu.async_copy(ft.at[srcv], rows, sem).wait()

        def grp(g, _2):
            sv = exv[pl.ds(g * 16, 16)]
            for e in range(16):
                bce = sv[jnp.full((16,), e, _i32)]
                r = g * 16 + e
                for j in range(F // 16):
                    sl = pl.ds(j * 16, 16)
                    rows[r, sl] = rows[r, sl] * bce
            return _2

        lax.fori_loop(0, CHUNKR // 16, grp, 0)
        pltpu.sync_copy(rows, acc.at[dstv], add=True)
        return _

    lax.fori_loop(0, nchunk, chunk, 0)
    plsc.subcore_barrier()
    pltpu.sync_copy(acc.at[pl.ds(s * seg, seg)],
                    out.at[pl.ds(c * NP + s * seg, seg)])


_BLK = 1000


def _tc_matmul_xw1(x, w1):
    def body(x_ref, w_ref, o_ref):
        o_ref[...] = jnp.dot(x_ref[...], w_ref[...],
                             preferred_element_type=_f32)

    return pl.pallas_call(
        body,
        grid=(N // _BLK,),
        in_specs=[
            pl.BlockSpec((_BLK, F), lambda i: (i, 0)),
            pl.BlockSpec((F, F), lambda i: (0, 0)),
        ],
        out_specs=pl.BlockSpec((_BLK, F), lambda i: (i, 0)),
        out_shape=jax.ShapeDtypeStruct((N, F), _f32),
    )(x, w1)


def _tc_norms(t1, d00, d01, d10, d11):
    def body(t1_ref, a_ref, b_ref, cc_ref, d_ref, t1n_ref, ns_ref, nd_ref):
        dego = a_ref[:, 0] + b_ref[:, 0]
        degi = cc_ref[:, 0] + d_ref[:, 0]
        ns = lax.rsqrt(jnp.maximum(dego, 1.0))
        nd = lax.rsqrt(jnp.maximum(degi, 1.0))
        t1n_ref[...] = t1_ref[...] * ns[:, None]
        ns_ref[:, 0] = ns
        nd_ref[:, 0] = nd

    vec = pl.BlockSpec((_BLK, 1), lambda i: (i, 0))
    return pl.pallas_call(
        body,
        grid=(N // _BLK,),
        in_specs=[pl.BlockSpec((_BLK, F), lambda i: (i, 0)), vec, vec, vec, vec],
        out_specs=[pl.BlockSpec((_BLK, F), lambda i: (i, 0)), vec, vec],
        out_shape=[
            jax.ShapeDtypeStruct((N, F), _f32),
            jax.ShapeDtypeStruct((N, 1), _f32),
            jax.ShapeDtypeStruct((N, 1), _f32),
        ],
    )(t1, d00, d01, d10, d11)


def _tc_gatprep(m0, m1, nd, b1, wg, al, ar):
    def body(m0_ref, m1_ref, nd_ref, b1_ref, wg_ref, al_ref, ar_ref,
             el_ref, er_ref, f2_ref):
        nd = nd_ref[:, 0]
        m = m0_ref[...] + m1_ref[...]
        h = m * nd[:, None] + b1_ref[0]
        nrm = jnp.sqrt(jnp.sum(h * h, axis=1, keepdims=True))
        h = h / jnp.maximum(nrm, 1e-12)
        h = jax.nn.sigmoid(h)
        feat = jnp.dot(h, wg_ref[...], preferred_element_type=_f32)
        cols = []
        for hh in range(HEADS):
            f = feat[:, hh * F:(hh + 1) * F]
            el = jnp.sum(f * al_ref[hh], axis=1)
            er = jnp.sum(f * ar_ref[hh], axis=1)
            f2_ref[hh] = f
            cols.append((el, er))
        (el0, er0), (el1, er1) = cols
        el_ref[...] = jnp.stack([el0, el1], axis=1)
        er_ref[...] = jnp.stack([er0, er1], axis=1)

    vec = pl.BlockSpec((_BLK, 1), lambda i: (i, 0))
    fullb = pl.BlockSpec((_BLK, F), lambda i: (i, 0))
    return pl.pallas_call(
        body,
        grid=(N // _BLK,),
        in_specs=[
            fullb, fullb, vec,
            pl.BlockSpec((1, F), lambda i: (0, 0)),
            pl.BlockSpec((F, HEADS * F), lambda i: (0, 0)),
            pl.BlockSpec((HEADS, F), lambda i: (0, 0)),
            pl.BlockSpec((HEADS, F), lambda i: (0, 0)),
        ],
        out_specs=[
            pl.BlockSpec((_BLK, 2), lambda i: (i, 0)),
            pl.BlockSpec((_BLK, 2), lambda i: (i, 0)),
            pl.BlockSpec((HEADS, _BLK, F), lambda i: (0, i, 0)),
        ],
        out_shape=[
            jax.ShapeDtypeStruct((N, 2), _f32),
            jax.ShapeDtypeStruct((N, 2), _f32),
            jax.ShapeDtypeStruct((HEADS, N, F), _f32),
        ],
    )(m0, m1, nd, b1, wg, al, ar)


_RBLK = 80


def _tc_readout(num0, num1, e00, e01, e10, e11, cap0, cap1, ns, bg, w2, b2,
                wc, bc):
    nblk = N // _RBLK

    def body(n0_ref, n1_ref, e00_ref, e01_ref, e10_ref, e11_ref,
             c0_ref, c1_ref, ns_ref, bg_ref, w2_ref, b2_ref, wc_ref, bc_ref,
             o_ref, wacc):
        i = pl.program_id(0)

        @pl.when(i == 0)
        def _():
            wacc[...] = jnp.zeros_like(wacc)

        cc = c0_ref[:, 0] + c1_ref[:, 0]
        sw = ns_ref[:, 0] * cc
        ws = []
        for hh, (n_ref, ea, eb) in enumerate(
                ((n0_ref, e00_ref, e10_ref), (n1_ref, e01_ref, e11_ref))):
            esum = ea[:, 0] + eb[:, 0]
            h2 = jax.nn.relu(n_ref[...] / (esum[:, None] + 1e-9) + bg_ref[hh])
            ws.append(jnp.dot(sw[None, :], h2, preferred_element_type=_f32))
        wacc[...] += jnp.concatenate(ws, axis=0)

        @pl.when(i == nblk - 1)
        def _():
            hg = jnp.dot(wacc[...] * (1.0 / N), w2_ref[...],
                         preferred_element_type=_f32) + b2_ref[0]
            o_ref[...] = jnp.dot(hg, wc_ref[...],
                                 preferred_element_type=_f32) + bc_ref[0]

    full = lambda shape: pl.BlockSpec(shape, lambda i: tuple(0 for _ in shape))
    rowb = pl.BlockSpec((_RBLK, F), lambda i: (i, 0))
    colb = pl.BlockSpec((_RBLK, 1), lambda i: (i, 0))
    return pl.pallas_call(
        body,
        grid=(nblk,),
        in_specs=[rowb, rowb, colb, colb, colb, colb, colb, colb, colb,
                  full((HEADS, F)), full((F, F)), full((1, F)),
                  full((F, 16)), full((1, 16))],
        out_specs=full((HEADS, 16)),
        out_shape=jax.ShapeDtypeStruct((HEADS, 16), _f32),
        scratch_shapes=[pltpu.VMEM((HEADS, F), _f32)],
    )(num0, num1, e00, e01, e10, e11, cap0, cap1, ns, bg, w2, b2, wc, bc)


def kernel(x, edge_index, W1, b1, Wg, al, ar, bg, W2, b2, Wc, bc):
    ei = edge_index.reshape(2 * E)
    z1 = jnp.zeros((NP,), _f32)
    z2 = jnp.zeros((NP, F), _f32)
    ones = jnp.ones((E // NWORK,), _f32)

    deg = _sc_degrees(ei, z1, ones)
    t1 = _tc_matmul_xw1(x, W1)

    dr = lambda k: deg[k * NP:k * NP + N].reshape(N, 1)
    t1n, ns, nd = _tc_norms(t1, dr(0), dr(2), dr(1), dr(3))

    mflat = _sc_conv1(ei, t1n, z2)
    m0 = mflat[:N]
    m1 = mflat[NP:NP + N]

    elt, ert, feat2 = _tc_gatprep(m0, m1, nd, b1.reshape(1, F), Wg, al, ar)

    exf, eap, cap = _sc_edgescalars(
        ei, elt[:, 0], elt[:, 1], ert[:, 0], ert[:, 1], nd.reshape(N), z1)

    ft = feat2.reshape(2 * N, F)
    numflat = _sc_gat(ei, exf, ft, z2)

    er = lambda k: eap[k * NP:(k + 1) * NP].reshape(NP, 1)
    out = _tc_readout(
        numflat[:N], numflat[NP:NP + N],
        er(0), er(1), er(2), er(3),
        cap[:NP].reshape(NP, 1), cap[NP:].reshape(NP, 1),
        ns, bg, W2, b2.reshape(1, F), Wc, bc.reshape(1, 16))
    return out.reshape(1, HEADS, 16)


# speedup vs baseline: 56.5945x; 1.1186x over previous
import jax
import jax.numpy as jnp
from jax import lax
from jax.experimental import pallas as pl
from jax.experimental.pallas import tpu as pltpu
from jax.experimental.pallas import tpu_sc as plsc

N = 10000
E = 320000
NP = 10240
HEADS = 2
F = 128
CHUNK = 400
CHUNKR = 80
NCORE = 2
NSUB = 16
NWORK = NCORE * NSUB

_mesh = lambda: plsc.VectorSubcoreMesh(
    core_axis_name="c", subcore_axis_name="s", num_cores=NCORE,
    num_subcores=NSUB)

_f32 = jnp.float32
_i32 = jnp.int32


def _iota16():
    return lax.iota(_i32, 16)


def _sc_degrees(ei, z1, ones):
    kfn = pl.kernel(
        _degrees_body,
        out_type=jax.ShapeDtypeStruct((4 * NP,), _f32),
        mesh=_mesh(),
        scratch_types=[
            pltpu.VMEM_SHARED((NP,), _f32),
            pltpu.VMEM_SHARED((NP,), _f32),
            pltpu.VMEM((E // NWORK,), _i32),
            pltpu.VMEM((E // NWORK,), _i32),
            pltpu.VMEM((E // NWORK,), _f32),
        ],
    )
    return kfn(ei, z1, ones)


def _degrees_body(ei, z1, ones, out, dego, degi, srcv, dstv, onesv):
    c = lax.axis_index("c")
    s = lax.axis_index("s")
    wid = c * NSUB + s
    per = E // NWORK
    base = wid * per
    seg = NP // NSUB
    pltpu.sync_copy(z1.at[pl.ds(s * seg, seg)], dego.at[pl.ds(s * seg, seg)])
    pltpu.sync_copy(z1.at[pl.ds(s * seg, seg)], degi.at[pl.ds(s * seg, seg)])
    pltpu.sync_copy(ei.at[pl.ds(base, per)], srcv)
    pltpu.sync_copy(ei.at[pl.ds(E + base, per)], dstv)
    pltpu.sync_copy(ones, onesv)
    plsc.subcore_barrier()
    pltpu.sync_copy(onesv, dego.at[srcv], add=True)
    pltpu.sync_copy(onesv, degi.at[dstv], add=True)
    plsc.subcore_barrier()
    pltpu.sync_copy(dego.at[pl.ds(s * seg, seg)],
                    out.at[pl.ds((c * 2 + 0) * NP + s * seg, seg)])
    pltpu.sync_copy(degi.at[pl.ds(s * seg, seg)],
                    out.at[pl.ds((c * 2 + 1) * NP + s * seg, seg)])


def _sc_conv1(ei, t1n, z2):
    kfn = pl.kernel(
        _conv1_body,
        out_type=jax.ShapeDtypeStruct((2 * NP, F), _f32),
        mesh=_mesh(),
        scratch_types=[
            pltpu.VMEM_SHARED((NP, F), _f32),
            pltpu.VMEM((CHUNKR,), _i32),
            pltpu.VMEM((CHUNKR,), _i32),
            pltpu.VMEM((CHUNKR, F), _f32),
            pltpu.VMEM((CHUNKR,), _i32),
            pltpu.VMEM((CHUNKR,), _i32),
            pltpu.VMEM((CHUNKR, F), _f32),
            pltpu.SemaphoreType.DMA,
            pltpu.SemaphoreType.DMA,
            pltpu.SemaphoreType.DMA,
            pltpu.SemaphoreType.DMA,
        ],
    )
    return kfn(ei, t1n, z2)


def _conv1_body(ei, t1n, z2, out, acc, srcA, dstA, rowsA, srcB, dstB, rowsB,
                gsA, gsB, ssA, ssB):
    c = lax.axis_index("c")
    s = lax.axis_index("s")
    seg = NP // NSUB
    pltpu.sync_copy(z2.at[pl.ds(s * seg, seg)], acc.at[pl.ds(s * seg, seg)])
    plsc.subcore_barrier()
    per = E // NWORK
    n = per // CHUNKR
    wid = c * NSUB + s
    slotA = (srcA, dstA, rowsA, gsA, ssA)
    slotB = (srcB, dstB, rowsB, gsB, ssB)

    def gstart(slot, ci):
        srcv, dstv, rows, gsem, _ = slot
        base = wid * per + ci * CHUNKR
        pltpu.sync_copy(ei.at[pl.ds(base, CHUNKR)], srcv)
        pltpu.sync_copy(ei.at[pl.ds(E + base, CHUNKR)], dstv)
        pltpu.async_copy(t1n.at[srcv], rows, gsem)

    def process(slot):
        srcv, dstv, rows, gsem, ssem = slot
        pltpu.make_async_copy(t1n.at[srcv], rows, gsem).wait()
        pltpu.async_copy(rows, acc.at[dstv], ssem, add=True)

    def sdone(slot):
        srcv, dstv, rows, _, ssem = slot
        pltpu.make_async_copy(rows, acc.at[dstv], ssem).wait()

    gstart(slotA, 0)
    gstart(slotB, 1)
    process(slotA)

    def pair(j, _):
        process(slotB)
        sdone(slotA)
        gstart(slotA, 2 * j + 2)
        process(slotA)
        sdone(slotB)
        gstart(slotB, 2 * j + 3)
        return _

    lax.fori_loop(0, (n - 2) // 2, pair, 0)
    process(slotB)
    sdone(slotA)
    gstart(slotA, n - 1)
    process(slotA)
    sdone(slotB)
    sdone(slotA)
    plsc.subcore_barrier()
    pltpu.sync_copy(acc.at[pl.ds(s * seg, seg)],
                    out.at[pl.ds(c * NP + s * seg, seg)])


def _sc_edgescalars(ei, el0, el1, er0, er1, ndt, z1):
    kfn = pl.kernel(
        _edgescalars_body,
        out_type=(
            jax.ShapeDtypeStruct((HEADS * E,), _f32),
            jax.ShapeDtypeStruct((4 * NP,), _f32),
            jax.ShapeDtypeStruct((2 * NP,), _f32),
        ),
        mesh=_mesh(),
        scratch_types=[
            pltpu.VMEM_SHARED((NP,), _f32),
            pltpu.VMEM_SHARED((NP,), _f32),
            pltpu.VMEM_SHARED((NP,), _f32),
            pltpu.VMEM((CHUNK,), _i32),
            pltpu.VMEM((CHUNK,), _i32),
            pltpu.VMEM((CHUNK,), _f32),
            pltpu.VMEM((CHUNK,), _f32),
            pltpu.VMEM((CHUNK,), _f32),
            pltpu.VMEM((CHUNK,), _f32),
            pltpu.VMEM((CHUNK,), _f32),
            pltpu.VMEM((CHUNK,), _f32),
            pltpu.VMEM((CHUNK,), _f32),
            pltpu.SemaphoreType.DMA,
        ],
    )
    return kfn(ei, el0, el1, er0, er1, ndt, z1)


def _edgescalars_body(ei, el0, el1, er0, er1, ndt, z1, ex_o, eap_o, cap_o,
                      eacc0, eacc1, cacc, srcv, dstv,
                      el0b, el1b, er0b, er1b, ndb, exb0, exb1, sem):
    c = lax.axis_index("c")
    s = lax.axis_index("s")
    wid = c * NSUB + s
    seg = NP // NSUB
    pltpu.sync_copy(z1.at[pl.ds(s * seg, seg)], eacc0.at[pl.ds(s * seg, seg)])
    pltpu.sync_copy(z1.at[pl.ds(s * seg, seg)], eacc1.at[pl.ds(s * seg, seg)])
    pltpu.sync_copy(z1.at[pl.ds(s * seg, seg)], cacc.at[pl.ds(s * seg, seg)])
    plsc.subcore_barrier()
    per = E // NWORK
    nchunk = per // CHUNK

    def chunk(i, _):
        base = wid * per + i * CHUNK
        pltpu.sync_copy(ei.at[pl.ds(base, CHUNK)], srcv)
        pltpu.sync_copy(ei.at[pl.ds(E + base, CHUNK)], dstv)
        ds = [pltpu.async_copy(el0.at[srcv], el0b, sem),
              pltpu.async_copy(el1.at[srcv], el1b, sem),
              pltpu.async_copy(er0.at[dstv], er0b, sem),
              pltpu.async_copy(er1.at[dstv], er1b, sem),
              pltpu.async_copy(ndt.at[dstv], ndb, sem)]
        for d in ds:
            d.wait()
        for g in range(CHUNK // 16):
            sl = pl.ds(g * 16, 16)
            for elb, erb, exb in ((el0b, er0b, exb0), (el1b, er1b, exb1)):
                t = elb[sl] + erb[sl]
                lr = jnp.where(t > 0.0, t, 0.2 * t)
                exb[sl] = jnp.exp(lr)
        pltpu.sync_copy(exb0, ex_o.at[pl.ds(base, CHUNK)])
        pltpu.sync_copy(exb1, ex_o.at[pl.ds(E + base, CHUNK)])
        pltpu.sync_copy(exb0, eacc0.at[dstv], add=True)
        pltpu.sync_copy(exb1, eacc1.at[dstv], add=True)
        pltpu.sync_copy(ndb, cacc.at[srcv], add=True)
        return _

    lax.fori_loop(0, nchunk, chunk, 0)
    plsc.subcore_barrier()
    pltpu.sync_copy(eacc0.at[pl.ds(s * seg, seg)],
                    eap_o.at[pl.ds((c * 2 + 0) * NP + s * seg, seg)])
    pltpu.sync_copy(eacc1.at[pl.ds(s * seg, seg)],
                    eap_o.at[pl.ds((c * 2 + 1) * NP + s * seg, seg)])
    pltpu.sync_copy(cacc.at[pl.ds(s * seg, seg)],
                    cap_o.at[pl.ds(c * NP + s * seg, seg)])


def _sc_gat(ei, exf, ft, z2):
    kfn = pl.kernel(
        _gat_body,
        out_type=jax.ShapeDtypeStruct((2 * NP, F), _f32),
        mesh=_mesh(),
        scratch_types=[
            pltpu.VMEM_SHARED((NP, F), _f32),
            pltpu.VMEM((CHUNKR,), _i32),
            pltpu.VMEM((CHUNKR,), _i32),
            pltpu.VMEM((CHUNKR,), _f32),
            pltpu.VMEM((CHUNKR, F), _f32),
            pltpu.VMEM((CHUNKR,), _i32),
            pltpu.VMEM((CHUNKR,), _i32),
            pltpu.VMEM((CHUNKR,), _f32),
            pltpu.VMEM((CHUNKR, F), _f32),
            pltpu.SemaphoreType.DMA,
            pltpu.SemaphoreType.DMA,
            pltpu.SemaphoreType.DMA,
            pltpu.SemaphoreType.DMA,
        ],
    )
    return kfn(ei, exf, ft, z2)


def _gat_body(ei, exf, ft, z2, out, acc,
              srcA, dstA, exA, rowsA, srcB, dstB, exB, rowsB,
              gsA, gsB, ssA, ssB):
    c = lax.axis_index("c")
    s = lax.axis_index("s")
    seg = NP // NSUB
    pltpu.sync_copy(z2.at[pl.ds(s * seg, seg)], acc.at[pl.ds(s * seg, seg)])
    plsc.subcore_barrier()
    per = E // NSUB
    n = per // CHUNKR
    delta = c * N
    slotA = (srcA, dstA, exA, rowsA, gsA, ssA)
    slotB = (srcB, dstB, exB, rowsB, gsB, ssB)

    def gstart(slot, ci):
        srcv, dstv, exv, rows, gsem, _ = slot
        base = s * per + ci * CHUNKR
        pltpu.sync_copy(ei.at[pl.ds(base, CHUNKR)], srcv)
        pltpu.sync_copy(ei.at[pl.ds(E + base, CHUNKR)], dstv)
        pltpu.sync_copy(exf.at[pl.ds(c * E + base, CHUNKR)], exv)
        for g in range(CHUNKR // 16):
            srcv[pl.ds(g * 16, 16)] = srcv[pl.ds(g * 16, 16)] + delta
        pltpu.async_copy(ft.at[srcv], rows, gsem)

    def process(slot):
        srcv, dstv, exv, rows, gsem, ssem = slot
        pltpu.make_async_copy(ft.at[srcv], rows, gsem).wait()

        def grp(g, _2):
            sv = exv[pl.ds(g * 16, 16)]
            for e in range(16):
                r = g * 16 + e
                ev = sv[e]
                for j in range(F // 16):
                    sl = pl.ds(j * 16, 16)
                    rows[r, sl] = rows[r, sl] * ev
            return _2

        lax.fori_loop(0, CHUNKR // 16, grp, 0)
        pltpu.async_copy(rows, acc.at[dstv], ssem, add=True)

    def sdone(slot):
        srcv, dstv, exv, rows, _, ssem = slot
        pltpu.make_async_copy(rows, acc.at[dstv], ssem).wait()

    gstart(slotA, 0)
    gstart(slotB, 1)
    process(slotA)

    def pair(j, _):
        process(slotB)
        sdone(slotA)
        gstart(slotA, 2 * j + 2)
        process(slotA)
        sdone(slotB)
        gstart(slotB, 2 * j + 3)
        return _

    lax.fori_loop(0, (n - 2) // 2, pair, 0)
    process(slotB)
    sdone(slotA)
    sdone(slotB)
    plsc.subcore_barrier()
    pltpu.sync_copy(acc.at[pl.ds(s * seg, seg)],
                    out.at[pl.ds(c * NP + s * seg, seg)])


_BLK = 1000


def _tc_matmul_xw1(x, w1):
    def body(x_ref, w_ref, o_ref):
        o_ref[...] = jnp.dot(x_ref[...], w_ref[...],
                             preferred_element_type=_f32)

    return pl.pallas_call(
        body,
        grid=(N // _BLK,),
        in_specs=[
            pl.BlockSpec((_BLK, F), lambda i: (i, 0)),
            pl.BlockSpec((F, F), lambda i: (0, 0)),
        ],
        out_specs=pl.BlockSpec((_BLK, F), lambda i: (i, 0)),
        out_shape=jax.ShapeDtypeStruct((N, F), _f32),
    )(x, w1)


def _tc_norms(t1, d00, d01, d10, d11):
    def body(t1_ref, a_ref, b_ref, cc_ref, d_ref, t1n_ref, ns_ref, nd_ref):
        dego = a_ref[:, 0] + b_ref[:, 0]
        degi = cc_ref[:, 0] + d_ref[:, 0]
        ns = lax.rsqrt(jnp.maximum(dego, 1.0))
        nd = lax.rsqrt(jnp.maximum(degi, 1.0))
        t1n_ref[...] = t1_ref[...] * ns[:, None]
        ns_ref[:, 0] = ns
        nd_ref[:, 0] = nd

    vec = pl.BlockSpec((_BLK, 1), lambda i: (i, 0))
    return pl.pallas_call(
        body,
        grid=(N // _BLK,),
        in_specs=[pl.BlockSpec((_BLK, F), lambda i: (i, 0)), vec, vec, vec, vec],
        out_specs=[pl.BlockSpec((_BLK, F), lambda i: (i, 0)), vec, vec],
        out_shape=[
            jax.ShapeDtypeStruct((N, F), _f32),
            jax.ShapeDtypeStruct((N, 1), _f32),
            jax.ShapeDtypeStruct((N, 1), _f32),
        ],
    )(t1, d00, d01, d10, d11)


def _tc_gatprep(m0, m1, nd, b1, wg, al, ar):
    def body(m0_ref, m1_ref, nd_ref, b1_ref, wg_ref, al_ref, ar_ref,
             el_ref, er_ref, f2_ref):
        nd = nd_ref[:, 0]
        m = m0_ref[...] + m1_ref[...]
        h = m * nd[:, None] + b1_ref[0]
        nrm = jnp.sqrt(jnp.sum(h * h, axis=1, keepdims=True))
        h = h / jnp.maximum(nrm, 1e-12)
        h = jax.nn.sigmoid(h)
        feat = jnp.dot(h, wg_ref[...], preferred_element_type=_f32)
        cols = []
        for hh in range(HEADS):
            f = feat[:, hh * F:(hh + 1) * F]
            el = jnp.sum(f * al_ref[hh], axis=1)
            er = jnp.sum(f * ar_ref[hh], axis=1)
            f2_ref[hh] = f
            cols.append((el, er))
        (el0, er0), (el1, er1) = cols
        el_ref[...] = jnp.stack([el0, el1], axis=1)
        er_ref[...] = jnp.stack([er0, er1], axis=1)

    vec = pl.BlockSpec((_BLK, 1), lambda i: (i, 0))
    fullb = pl.BlockSpec((_BLK, F), lambda i: (i, 0))
    return pl.pallas_call(
        body,
        grid=(N // _BLK,),
        in_specs=[
            fullb, fullb, vec,
            pl.BlockSpec((1, F), lambda i: (0, 0)),
            pl.BlockSpec((F, HEADS * F), lambda i: (0, 0)),
            pl.BlockSpec((HEADS, F), lambda i: (0, 0)),
            pl.BlockSpec((HEADS, F), lambda i: (0, 0)),
        ],
        out_specs=[
            pl.BlockSpec((_BLK, 2), lambda i: (i, 0)),
            pl.BlockSpec((_BLK, 2), lambda i: (i, 0)),
            pl.BlockSpec((HEADS, _BLK, F), lambda i: (0, i, 0)),
        ],
        out_shape=[
            jax.ShapeDtypeStruct((N, 2), _f32),
            jax.ShapeDtypeStruct((N, 2), _f32),
            jax.ShapeDtypeStruct((HEADS, N, F), _f32),
        ],
    )(m0, m1, nd, b1, wg, al, ar)


_RBLK = 80


def _tc_readout(num0, num1, e00, e01, e10, e11, cap0, cap1, ns, bg, w2, b2,
                wc, bc):
    nblk = N // _RBLK

    def body(n0_ref, n1_ref, e00_ref, e01_ref, e10_ref, e11_ref,
             c0_ref, c1_ref, ns_ref, bg_ref, w2_ref, b2_ref, wc_ref, bc_ref,
             o_ref, wacc):
        i = pl.program_id(0)

        @pl.when(i == 0)
        def _():
            wacc[...] = jnp.zeros_like(wacc)

        cc = c0_ref[:, 0] + c1_ref[:, 0]
        sw = ns_ref[:, 0] * cc
        ws = []
        for hh, (n_ref, ea, eb) in enumerate(
                ((n0_ref, e00_ref, e10_ref), (n1_ref, e01_ref, e11_ref))):
            esum = ea[:, 0] + eb[:, 0]
            h2 = jax.nn.relu(n_ref[...] / (esum[:, None] + 1e-9) + bg_ref[hh])
            ws.append(jnp.dot(sw[None, :], h2, preferred_element_type=_f32))
        wacc[...] += jnp.concatenate(ws, axis=0)

        @pl.when(i == nblk - 1)
        def _():
            hg = jnp.dot(wacc[...] * (1.0 / N), w2_ref[...],
                         preferred_element_type=_f32) + b2_ref[0]
            o_ref[...] = jnp.dot(hg, wc_ref[...],
                                 preferred_element_type=_f32) + bc_ref[0]

    full = lambda shape: pl.BlockSpec(shape, lambda i: tuple(0 for _ in shape))
    rowb = pl.BlockSpec((_RBLK, F), lambda i: (i, 0))
    colb = pl.BlockSpec((_RBLK, 1), lambda i: (i, 0))
    return pl.pallas_call(
        body,
        grid=(nblk,),
        in_specs=[rowb, rowb, colb, colb, colb, colb, colb, colb, colb,
                  full((HEADS, F)), full((F, F)), full((1, F)),
                  full((F, 16)), full((1, 16))],
        out_specs=full((HEADS, 16)),
        out_shape=jax.ShapeDtypeStruct((HEADS, 16), _f32),
        scratch_shapes=[pltpu.VMEM((HEADS, F), _f32)],
    )(num0, num1, e00, e01, e10, e11, cap0, cap1, ns, bg, w2, b2, wc, bc)


def kernel(x, edge_index, W1, b1, Wg, al, ar, bg, W2, b2, Wc, bc):
    ei = edge_index.reshape(2 * E)
    z1 = jnp.zeros((NP,), _f32)
    z2 = jnp.zeros((NP, F), _f32)
    ones = jnp.ones((E // NWORK,), _f32)

    deg = _sc_degrees(ei, z1, ones)
    t1 = _tc_matmul_xw1(x, W1)

    dr = lambda k: deg[k * NP:k * NP + N].reshape(N, 1)
    t1n, ns, nd = _tc_norms(t1, dr(0), dr(2), dr(1), dr(3))

    mflat = _sc_conv1(ei, t1n, z2)
    m0 = mflat[:N]
    m1 = mflat[NP:NP + N]

    elt, ert, feat2 = _tc_gatprep(m0, m1, nd, b1.reshape(1, F), Wg, al, ar)

    exf, eap, cap = _sc_edgescalars(
        ei, elt[:, 0], elt[:, 1], ert[:, 0], ert[:, 1], nd.reshape(N), z1)

    ft = feat2.reshape(2 * N, F)
    numflat = _sc_gat(ei, exf, ft, z2)

    er = lambda k: eap[k * NP:(k + 1) * NP].reshape(NP, 1)
    out = _tc_readout(
        numflat[:N], numflat[NP:NP + N],
        er(0), er(1), er(2), er(3),
        cap[:NP].reshape(NP, 1), cap[NP:].reshape(NP, 1),
        ns, bg, W2, b2.reshape(1, F), Wc, bc.reshape(1, 16))
    return out.reshape(1, HEADS, 16)


# speedup vs baseline: 81.3549x; 1.4375x over previous
import jax
import jax.numpy as jnp
from jax import lax
from jax.experimental import pallas as pl
from jax.experimental.pallas import tpu as pltpu
from jax.experimental.pallas import tpu_sc as plsc

N = 10000
E = 320000
NP = 10240
HEADS = 2
F = 128
CH = 128
NREC = 2560
EP = NREC * CH
NCORE = 2
NSUB = 16
NWORK = NCORE * NSUB

_mesh = lambda: plsc.VectorSubcoreMesh(
    core_axis_name="c", subcore_axis_name="s", num_cores=NCORE,
    num_subcores=NSUB)

_f32 = jnp.float32
_i32 = jnp.int32


def _sc_degrees(ei, z1, ones):
    kfn = pl.kernel(
        _degrees_body,
        out_type=jax.ShapeDtypeStruct((4 * NP,), _f32),
        mesh=_mesh(),
        scratch_types=[
            pltpu.VMEM_SHARED((NP,), _f32),
            pltpu.VMEM_SHARED((NP,), _f32),
            pltpu.VMEM((E // NWORK,), _i32),
            pltpu.VMEM((E // NWORK,), _i32),
            pltpu.VMEM((E // NWORK,), _f32),
        ],
    )
    return kfn(ei, z1, ones)


def _degrees_body(ei, z1, ones, out, dego, degi, srcv, dstv, onesv):
    c = lax.axis_index("c")
    s = lax.axis_index("s")
    wid = c * NSUB + s
    per = E // NWORK
    base = wid * per
    seg = NP // NSUB
    pltpu.sync_copy(z1.at[pl.ds(s * seg, seg)], dego.at[pl.ds(s * seg, seg)])
    pltpu.sync_copy(z1.at[pl.ds(s * seg, seg)], degi.at[pl.ds(s * seg, seg)])
    pltpu.sync_copy(ei.at[pl.ds(base, per)], srcv)
    pltpu.sync_copy(ei.at[pl.ds(E + base, per)], dstv)
    pltpu.sync_copy(ones, onesv)
    plsc.subcore_barrier()
    pltpu.sync_copy(onesv, dego.at[srcv], add=True)
    pltpu.sync_copy(onesv, degi.at[dstv], add=True)
    plsc.subcore_barrier()
    pltpu.sync_copy(dego.at[pl.ds(s * seg, seg)],
                    out.at[pl.ds((c * 2 + 0) * NP + s * seg, seg)])
    pltpu.sync_copy(degi.at[pl.ds(s * seg, seg)],
                    out.at[pl.ds((c * 2 + 1) * NP + s * seg, seg)])


def _pipeline(gstart, process, sdone, slotA, slotB, K):
    gstart(slotA, 0)
    gstart(slotB, 1)
    process(slotA, 0)

    def pair(j, _):
        process(slotB, 2 * j + 1)
        sdone(slotA, 2 * j)
        gstart(slotA, 2 * j + 2)
        process(slotA, 2 * j + 2)
        sdone(slotB, 2 * j + 1)
        gstart(slotB, 2 * j + 3)
        return _

    lax.fori_loop(0, (K - 2) // 2, pair, 0)
    if K % 2:
        process(slotB, K - 2)
        sdone(slotA, K - 3)
        gstart(slotA, K - 1)
        process(slotA, K - 1)
        sdone(slotB, K - 2)
        sdone(slotA, K - 1)
    else:
        process(slotB, K - 1)
        sdone(slotA, K - 2)
        sdone(slotB, K - 1)


_KB = 16


def _sc_conv1(eips, eipd, t1np, z2):
    kfn = pl.kernel(
        _conv1_body,
        out_type=jax.ShapeDtypeStruct((2 * NP, F), _f32),
        mesh=_mesh(),
        scratch_types=[
            pltpu.VMEM_SHARED((NP, F), _f32),
            pltpu.VMEM((_KB, CH), _i32),
            pltpu.VMEM((_KB, CH), _i32),
            pltpu.VMEM((CH, F), _f32),
            pltpu.VMEM((CH,), _i32),
            pltpu.VMEM((CH,), _i32),
            pltpu.VMEM((CH, F), _f32),
            pltpu.VMEM((CH,), _i32),
            pltpu.VMEM((CH,), _i32),
            pltpu.SemaphoreType.DMA,
            pltpu.SemaphoreType.DMA,
            pltpu.SemaphoreType.DMA,
            pltpu.SemaphoreType.DMA,
        ],
    )
    return kfn(eips, eipd, t1np, z2)


def _conv1_body(eips, eipd, t1np, z2, out, acc, sarena, darena,
                rowsA, srcbA, dstbA, rowsB, srcbB, dstbB,
                gsA, gsB, ssA, ssB):
    c = lax.axis_index("c")
    s = lax.axis_index("s")
    seg = NP // NSUB
    pltpu.sync_copy(z2.at[pl.ds(s * seg, seg)], acc.at[pl.ds(s * seg, seg)])
    plsc.subcore_barrier()
    nrec = NREC // NWORK
    nsup = nrec // _KB
    wid = c * NSUB + s
    slotA = (rowsA, srcbA, dstbA, gsA, ssA)
    slotB = (rowsB, srcbB, dstbB, gsB, ssB)

    def gstart(slot, k):
        rows, srcb, dstb, gsem, _ = slot
        for g in range(CH // 16):
            sl = pl.ds(g * 16, 16)
            srcb[sl] = sarena[k, sl]
            dstb[sl] = darena[k, sl]
        pltpu.async_copy(t1np.at[srcb], rows, gsem)

    def process(slot, k):
        rows, srcb, dstb, gsem, ssem = slot
        pltpu.make_async_copy(t1np.at[srcb], rows, gsem).wait()
        pltpu.async_copy(rows, acc.at[dstb], ssem, add=True)

    def sdone(slot, k):
        rows, srcb, dstb, _, ssem = slot
        pltpu.make_async_copy(rows, acc.at[dstb], ssem).wait()

    def sup(sp, _):
        rec = wid * nrec + sp * _KB
        pltpu.sync_copy(eips.at[pl.ds(rec, _KB)], sarena)
        pltpu.sync_copy(eipd.at[pl.ds(rec, _KB)], darena)
        _pipeline(gstart, process, sdone, slotA, slotB, _KB)
        return _

    lax.fori_loop(0, nsup, sup, 0)
    plsc.subcore_barrier()
    pltpu.sync_copy(acc.at[pl.ds(s * seg, seg)],
                    out.at[pl.ds(c * NP + s * seg, seg)])


_KC = 80


def _sc_edgescalars(eips, eipd, el0, el1, er0, er1, ndt, z1):
    def slot_bufs():
        return [pltpu.VMEM((CH,), _f32) for _ in range(7)] + [
            pltpu.VMEM((CH,), _i32), pltpu.VMEM((CH,), _i32),
            pltpu.SemaphoreType.DMA, pltpu.SemaphoreType.DMA]

    kfn = pl.kernel(
        _edgescalars_body,
        out_type=(
            jax.ShapeDtypeStruct((HEADS * EP,), _f32),
            jax.ShapeDtypeStruct((4 * NP,), _f32),
            jax.ShapeDtypeStruct((2 * NP,), _f32),
        ),
        mesh=_mesh(),
        scratch_types=[
            pltpu.VMEM_SHARED((NP,), _f32),
            pltpu.VMEM_SHARED((NP,), _f32),
            pltpu.VMEM_SHARED((NP,), _f32),
            pltpu.VMEM((_KC, CH), _i32),
            pltpu.VMEM((_KC, CH), _i32),
        ] + slot_bufs() + slot_bufs(),
    )
    return kfn(eips, eipd, el0, el1, er0, er1, ndt, z1)


def _edgescalars_body(eips, eipd, el0, el1, er0, er1, ndt, z1,
                      ex_o, eap_o, cap_o,
                      eacc0, eacc1, cacc, sarena, darena, *slots):
    c = lax.axis_index("c")
    s = lax.axis_index("s")
    wid = c * NSUB + s
    seg = NP // NSUB
    pltpu.sync_copy(z1.at[pl.ds(s * seg, seg)], eacc0.at[pl.ds(s * seg, seg)])
    pltpu.sync_copy(z1.at[pl.ds(s * seg, seg)], eacc1.at[pl.ds(s * seg, seg)])
    pltpu.sync_copy(z1.at[pl.ds(s * seg, seg)], cacc.at[pl.ds(s * seg, seg)])
    plsc.subcore_barrier()
    slotA = slots[:11]
    slotB = slots[11:]
    tabs = (el0, el1, er0, er1, ndt)

    def bufs_of(slot, k):
        srcb, dstb = slot[7], slot[8]
        idx = [srcb, srcb, dstb, dstb, dstb]
        return list(zip(tabs, idx, slot[:5]))

    def gstart(slot, k):
        srcb, dstb, gsem = slot[7], slot[8], slot[9]
        for g in range(CH // 16):
            sl = pl.ds(g * 16, 16)
            srcb[sl] = sarena[k, sl]
            dstb[sl] = darena[k, sl]
        for tab, idx, buf in bufs_of(slot, k):
            pltpu.async_copy(tab.at[idx], buf, gsem)

    def process(slot, k):
        el0b, el1b, er0b, er1b, ndb, exb0, exb1, srcb, dstb, gsem, ssem = slot
        for tab, idx, buf in bufs_of(slot, k):
            pltpu.make_async_copy(tab.at[idx], buf, gsem).wait()
        for g in range(CH // 16):
            sl = pl.ds(g * 16, 16)
            for elb, erb, exb in ((el0b, er0b, exb0), (el1b, er1b, exb1)):
                t = elb[sl] + erb[sl]
                lr = jnp.where(t > 0.0, t, 0.2 * t)
                exb[sl] = jnp.exp(lr)
        base = (wid * _KC + k) * CH
        pltpu.sync_copy(exb0, ex_o.at[pl.ds(base, CH)])
        pltpu.sync_copy(exb1, ex_o.at[pl.ds(EP + base, CH)])
        pltpu.sync_copy(exb0, eacc0.at[dstb], add=True)
        pltpu.sync_copy(exb1, eacc1.at[dstb], add=True)
        pltpu.sync_copy(ndb, cacc.at[srcb], add=True)

    def sdone(slot, k):
        pass

    pltpu.sync_copy(eips.at[pl.ds(wid * _KC, _KC)], sarena)
    pltpu.sync_copy(eipd.at[pl.ds(wid * _KC, _KC)], darena)
    _pipeline(gstart, process, sdone, slotA, slotB, _KC)
    plsc.subcore_barrier()
    pltpu.sync_copy(eacc0.at[pl.ds(s * seg, seg)],
                    eap_o.at[pl.ds((c * 2 + 0) * NP + s * seg, seg)])
    pltpu.sync_copy(eacc1.at[pl.ds(s * seg, seg)],
                    eap_o.at[pl.ds((c * 2 + 1) * NP + s * seg, seg)])
    pltpu.sync_copy(cacc.at[pl.ds(s * seg, seg)],
                    cap_o.at[pl.ds(c * NP + s * seg, seg)])


_KD = 16


def _sc_gat(eips, eipd, ex3, ftp, z2):
    kfn = pl.kernel(
        _gat_body,
        out_type=jax.ShapeDtypeStruct((2 * NP, F), _f32),
        mesh=_mesh(),
        scratch_types=[
            pltpu.VMEM_SHARED((NP, F), _f32),
            pltpu.VMEM((_KD, CH), _i32),
            pltpu.VMEM((_KD, CH), _i32),
            pltpu.VMEM((_KD, CH), _f32),
            pltpu.VMEM((CH, F), _f32),
            pltpu.VMEM((CH,), _i32),
            pltpu.VMEM((CH,), _i32),
            pltpu.VMEM((CH, F), _f32),
            pltpu.VMEM((CH,), _i32),
            pltpu.VMEM((CH,), _i32),
            pltpu.SemaphoreType.DMA,
            pltpu.SemaphoreType.DMA,
            pltpu.SemaphoreType.DMA,
            pltpu.SemaphoreType.DMA,
        ],
    )
    return kfn(eips, eipd, ex3, ftp, z2)


def _gat_body(eips, eipd, ex3, ftp, z2, out, acc, sarena, darena, exarena,
              rowsA, srcbA, dstbA, rowsB, srcbB, dstbB,
              gsA, gsB, ssA, ssB):
    c = lax.axis_index("c")
    s = lax.axis_index("s")
    seg = NP // NSUB
    pltpu.sync_copy(z2.at[pl.ds(s * seg, seg)], acc.at[pl.ds(s * seg, seg)])
    plsc.subcore_barrier()
    nrec = NREC // NSUB
    nsup = nrec // _KD
    delta = c * NP
    slotA = (rowsA, srcbA, dstbA, gsA, ssA)
    slotB = (rowsB, srcbB, dstbB, gsB, ssB)

    def gstart(slot, k):
        rows, srcb, dstb, gsem, _ = slot
        for g in range(CH // 16):
            sl = pl.ds(g * 16, 16)
            srcb[sl] = sarena[k, sl] + delta
            dstb[sl] = darena[k, sl]
        pltpu.async_copy(ftp.at[srcb], rows, gsem)

    def process(slot, k):
        rows, srcb, dstb, gsem, ssem = slot
        pltpu.make_async_copy(ftp.at[srcb], rows, gsem).wait()

        def grp(g, _2):
            sv = exarena[k, pl.ds(g * 16, 16)]
            for e in range(16):
                r = g * 16 + e
                ev = sv[e]
                for j in range(F // 16):
                    sl = pl.ds(j * 16, 16)
                    rows[r, sl] = rows[r, sl] * ev
            return _2

        lax.fori_loop(0, CH // 16, grp, 0)
        pltpu.async_copy(rows, acc.at[dstb], ssem, add=True)

    def sdone(slot, k):
        rows, srcb, dstb, _, ssem = slot
        pltpu.make_async_copy(rows, acc.at[dstb], ssem).wait()

    def sup(sp, _):
        rec = s * nrec + sp * _KD
        pltpu.sync_copy(eips.at[pl.ds(rec, _KD)], sarena)
        pltpu.sync_copy(eipd.at[pl.ds(rec, _KD)], darena)
        exblk = c * (NREC // _KD) + s * nsup + sp
        pltpu.sync_copy(ex3.at[exblk], exarena)
        _pipeline(gstart, process, sdone, slotA, slotB, _KD)
        return _

    lax.fori_loop(0, nsup, sup, 0)
    plsc.subcore_barrier()
    pltpu.sync_copy(acc.at[pl.ds(s * seg, seg)],
                    out.at[pl.ds(c * NP + s * seg, seg)])


_BLK = 1000


def _tc_matmul_xw1(x, w1):
    def body(x_ref, w_ref, o_ref):
        o_ref[...] = jnp.dot(x_ref[...], w_ref[...],
                             preferred_element_type=_f32)

    return pl.pallas_call(
        body,
        grid=(N // _BLK,),
        in_specs=[
            pl.BlockSpec((_BLK, F), lambda i: (i, 0)),
            pl.BlockSpec((F, F), lambda i: (0, 0)),
        ],
        out_specs=pl.BlockSpec((_BLK, F), lambda i: (i, 0)),
        out_shape=jax.ShapeDtypeStruct((N, F), _f32),
    )(x, w1)


def _tc_norms(t1, d00, d01, d10, d11):
    def body(t1_ref, a_ref, b_ref, cc_ref, d_ref, t1n_ref, ns_ref, nd_ref):
        dego = a_ref[:, 0] + b_ref[:, 0]
        degi = cc_ref[:, 0] + d_ref[:, 0]
        ns = lax.rsqrt(jnp.maximum(dego, 1.0))
        nd = lax.rsqrt(jnp.maximum(degi, 1.0))
        t1n_ref[...] = t1_ref[...] * ns[:, None]
        ns_ref[:, 0] = ns
        nd_ref[:, 0] = nd

    vec = pl.BlockSpec((_BLK, 1), lambda i: (i, 0))
    return pl.pallas_call(
        body,
        grid=(N // _BLK,),
        in_specs=[pl.BlockSpec((_BLK, F), lambda i: (i, 0)), vec, vec, vec, vec],
        out_specs=[pl.BlockSpec((_BLK, F), lambda i: (i, 0)), vec, vec],
        out_shape=[
            jax.ShapeDtypeStruct((N, F), _f32),
            jax.ShapeDtypeStruct((N, 1), _f32),
            jax.ShapeDtypeStruct((N, 1), _f32),
        ],
    )(t1, d00, d01, d10, d11)


def _tc_gatprep(m0, m1, nd, b1, wg, al, ar):
    def body(m0_ref, m1_ref, nd_ref, b1_ref, wg_ref, al_ref, ar_ref,
             el_ref, er_ref, f2_ref):
        nd = nd_ref[:, 0]
        m = m0_ref[...] + m1_ref[...]
        h = m * nd[:, None] + b1_ref[0]
        nrm = jnp.sqrt(jnp.sum(h * h, axis=1, keepdims=True))
        h = h / jnp.maximum(nrm, 1e-12)
        h = jax.nn.sigmoid(h)
        feat = jnp.dot(h, wg_ref[...], preferred_element_type=_f32)
        cols = []
        for hh in range(HEADS):
            f = feat[:, hh * F:(hh + 1) * F]
            el = jnp.sum(f * al_ref[hh], axis=1)
            er = jnp.sum(f * ar_ref[hh], axis=1)
            f2_ref[hh] = f
            cols.append((el, er))
        (el0, er0), (el1, er1) = cols
        el_ref[...] = jnp.stack([el0, el1], axis=1)
        er_ref[...] = jnp.stack([er0, er1], axis=1)

    vec = pl.BlockSpec((_BLK, 1), lambda i: (i, 0))
    fullb = pl.BlockSpec((_BLK, F), lambda i: (i, 0))
    return pl.pallas_call(
        body,
        grid=(N // _BLK,),
        in_specs=[
            fullb, fullb, vec,
            pl.BlockSpec((1, F), lambda i: (0, 0)),
            pl.BlockSpec((F, HEADS * F), lambda i: (0, 0)),
            pl.BlockSpec((HEADS, F), lambda i: (0, 0)),
            pl.BlockSpec((HEADS, F), lambda i: (0, 0)),
        ],
        out_specs=[
            pl.BlockSpec((_BLK, 2), lambda i: (i, 0)),
            pl.BlockSpec((_BLK, 2), lambda i: (i, 0)),
            pl.BlockSpec((HEADS, _BLK, F), lambda i: (0, i, 0)),
        ],
        out_shape=[
            jax.ShapeDtypeStruct((N, 2), _f32),
            jax.ShapeDtypeStruct((N, 2), _f32),
            jax.ShapeDtypeStruct((HEADS, N, F), _f32),
        ],
    )(m0, m1, nd, b1, wg, al, ar)


_RBLK = 80


def _tc_readout(num0, num1, e00, e01, e10, e11, cap0, cap1, ns, bg, w2, b2,
                wc, bc):
    nblk = N // _RBLK

    def body(n0_ref, n1_ref, e00_ref, e01_ref, e10_ref, e11_ref,
             c0_ref, c1_ref, ns_ref, bg_ref, w2_ref, b2_ref, wc_ref, bc_ref,
             o_ref, wacc):
        i = pl.program_id(0)

        @pl.when(i == 0)
        def _():
            wacc[...] = jnp.zeros_like(wacc)

        cc = c0_ref[:, 0] + c1_ref[:, 0]
        sw = ns_ref[:, 0] * cc
        ws = []
        for hh, (n_ref, ea, eb) in enumerate(
                ((n0_ref, e00_ref, e10_ref), (n1_ref, e01_ref, e11_ref))):
            esum = ea[:, 0] + eb[:, 0]
            h2 = jax.nn.relu(n_ref[...] / (esum[:, None] + 1e-9) + bg_ref[hh])
            ws.append(jnp.dot(sw[None, :], h2, preferred_element_type=_f32))
        wacc[...] += jnp.concatenate(ws, axis=0)

        @pl.when(i == nblk - 1)
        def _():
            hg = jnp.dot(wacc[...] * (1.0 / N), w2_ref[...],
                         preferred_element_type=_f32) + b2_ref[0]
            o_ref[...] = jnp.dot(hg, wc_ref[...],
                                 preferred_element_type=_f32) + bc_ref[0]

    full = lambda shape: pl.BlockSpec(shape, lambda i: tuple(0 for _ in shape))
    rowb = pl.BlockSpec((_RBLK, F), lambda i: (i, 0))
    colb = pl.BlockSpec((_RBLK, 1), lambda i: (i, 0))
    return pl.pallas_call(
        body,
        grid=(nblk,),
        in_specs=[rowb, rowb, colb, colb, colb, colb, colb, colb, colb,
                  full((HEADS, F)), full((F, F)), full((1, F)),
                  full((F, 16)), full((1, 16))],
        out_specs=full((HEADS, 16)),
        out_shape=jax.ShapeDtypeStruct((HEADS, 16), _f32),
        scratch_shapes=[pltpu.VMEM((HEADS, F), _f32)],
    )(num0, num1, e00, e01, e10, e11, cap0, cap1, ns, bg, w2, b2, wc, bc)


def kernel(x, edge_index, W1, b1, Wg, al, ar, bg, W2, b2, Wc, bc):
    ei = edge_index.reshape(2 * E)
    pad = EP - E
    padidx = (N + (jnp.arange(pad, dtype=_i32) % (NP - N))).astype(_i32)
    srcp = jnp.concatenate([edge_index[0], padidx])
    dstp = jnp.concatenate([edge_index[1], padidx])
    eips = srcp.reshape(NREC, CH)
    eipd = dstp.reshape(NREC, CH)
    z1 = jnp.zeros((NP,), _f32)
    z2 = jnp.zeros((NP, F), _f32)
    ones = jnp.ones((E // NWORK,), _f32)
    zpadrow = jnp.zeros((NP - N, F), _f32)
    zpad = jnp.zeros((NP - N,), _f32)

    deg = _sc_degrees(ei, z1, ones)
    t1 = _tc_matmul_xw1(x, W1)

    dr = lambda k: deg[k * NP:k * NP + N].reshape(N, 1)
    t1n, ns, nd = _tc_norms(t1, dr(0), dr(2), dr(1), dr(3))

    t1np = jnp.concatenate([t1n, zpadrow])
    mflat = _sc_conv1(eips, eipd, t1np, z2)
    m0 = mflat[:N]
    m1 = mflat[NP:NP + N]

    elt, ert, feat2 = _tc_gatprep(m0, m1, nd, b1.reshape(1, F), Wg, al, ar)

    tpad = lambda v: jnp.concatenate([v, zpad])
    exf, eap, cap = _sc_edgescalars(
        eips, eipd, tpad(elt[:, 0]), tpad(elt[:, 1]), tpad(ert[:, 0]),
        tpad(ert[:, 1]), tpad(nd.reshape(N)), z1)

    ftp = jnp.concatenate([feat2[0], zpadrow, feat2[1], zpadrow])
    ex3 = exf.reshape(2 * NREC // _KD, _KD, CH)
    numflat = _sc_gat(eips, eipd, ex3, ftp, z2)

    er = lambda k: eap[k * NP:(k + 1) * NP].reshape(NP, 1)
    out = _tc_readout(
        numflat[:N], numflat[NP:NP + N],
        er(0), er(1), er(2), er(3),
        cap[:NP].reshape(NP, 1), cap[NP:].reshape(NP, 1),
        ns, bg, W2, b2.reshape(1, F), Wc, bc.reshape(1, 16))
    return out.reshape(1, HEADS, 16)


# speedup vs baseline: 84.2606x; 1.0357x over previous
import jax
import jax.numpy as jnp
from jax import lax
from jax.experimental import pallas as pl
from jax.experimental.pallas import tpu as pltpu
from jax.experimental.pallas import tpu_sc as plsc

N = 10000
E = 320000
NP = 10240
HEADS = 2
F = 128
CH = 128
NREC = 2560
EP = NREC * CH
NCORE = 2
NSUB = 16
NWORK = NCORE * NSUB

_mesh = lambda: plsc.VectorSubcoreMesh(
    core_axis_name="c", subcore_axis_name="s", num_cores=NCORE,
    num_subcores=NSUB)

_f32 = jnp.float32
_i32 = jnp.int32


def _sc_degrees(ei, z1, ones):
    kfn = pl.kernel(
        _degrees_body,
        out_type=jax.ShapeDtypeStruct((4 * NP,), _f32),
        mesh=_mesh(),
        scratch_types=[
            pltpu.VMEM_SHARED((NP,), _f32),
            pltpu.VMEM_SHARED((NP,), _f32),
            pltpu.VMEM((E // NWORK,), _i32),
            pltpu.VMEM((E // NWORK,), _i32),
            pltpu.VMEM((E // NWORK,), _f32),
        ],
    )
    return kfn(ei, z1, ones)


def _degrees_body(ei, z1, ones, out, dego, degi, srcv, dstv, onesv):
    c = lax.axis_index("c")
    s = lax.axis_index("s")
    wid = c * NSUB + s
    per = E // NWORK
    base = wid * per
    seg = NP // NSUB
    pltpu.sync_copy(z1.at[pl.ds(s * seg, seg)], dego.at[pl.ds(s * seg, seg)])
    pltpu.sync_copy(z1.at[pl.ds(s * seg, seg)], degi.at[pl.ds(s * seg, seg)])
    pltpu.sync_copy(ei.at[pl.ds(base, per)], srcv)
    pltpu.sync_copy(ei.at[pl.ds(E + base, per)], dstv)
    pltpu.sync_copy(ones, onesv)
    plsc.subcore_barrier()
    pltpu.sync_copy(onesv, dego.at[srcv], add=True)
    pltpu.sync_copy(onesv, degi.at[dstv], add=True)
    plsc.subcore_barrier()
    pltpu.sync_copy(dego.at[pl.ds(s * seg, seg)],
                    out.at[pl.ds((c * 2 + 0) * NP + s * seg, seg)])
    pltpu.sync_copy(degi.at[pl.ds(s * seg, seg)],
                    out.at[pl.ds((c * 2 + 1) * NP + s * seg, seg)])


def _pipeline(gstart, process, sdone, slotA, slotB, K):
    gstart(slotA, 0)
    gstart(slotB, 1)
    process(slotA, 0)

    def pair(j, _):
        process(slotB, 2 * j + 1)
        sdone(slotA, 2 * j)
        gstart(slotA, 2 * j + 2)
        process(slotA, 2 * j + 2)
        sdone(slotB, 2 * j + 1)
        gstart(slotB, 2 * j + 3)
        return _

    lax.fori_loop(0, (K - 2) // 2, pair, 0)
    if K % 2:
        process(slotB, K - 2)
        sdone(slotA, K - 3)
        gstart(slotA, K - 1)
        process(slotA, K - 1)
        sdone(slotB, K - 2)
        sdone(slotA, K - 1)
    else:
        process(slotB, K - 1)
        sdone(slotA, K - 2)
        sdone(slotB, K - 1)


_KB = 16


def _sc_conv1(eips, eipd, t1np, z2):
    kfn = pl.kernel(
        _conv1_body,
        out_type=jax.ShapeDtypeStruct((2 * NP, F), _f32),
        mesh=_mesh(),
        scratch_types=[
            pltpu.VMEM_SHARED((NP, F), _f32),
            pltpu.VMEM((_KB, CH), _i32),
            pltpu.VMEM((_KB, CH), _i32),
            pltpu.VMEM((CH, F), _f32),
            pltpu.VMEM((CH,), _i32),
            pltpu.VMEM((CH,), _i32),
            pltpu.VMEM((CH, F), _f32),
            pltpu.VMEM((CH,), _i32),
            pltpu.VMEM((CH,), _i32),
            pltpu.SemaphoreType.DMA,
            pltpu.SemaphoreType.DMA,
            pltpu.SemaphoreType.DMA,
            pltpu.SemaphoreType.DMA,
        ],
    )
    return kfn(eips, eipd, t1np, z2)


def _conv1_body(eips, eipd, t1np, z2, out, acc, sarena, darena,
                rowsA, srcbA, dstbA, rowsB, srcbB, dstbB,
                gsA, gsB, ssA, ssB):
    c = lax.axis_index("c")
    s = lax.axis_index("s")
    seg = NP // NSUB
    pltpu.sync_copy(z2.at[pl.ds(s * seg, seg)], acc.at[pl.ds(s * seg, seg)])
    plsc.subcore_barrier()
    nrec = NREC // NWORK
    nsup = nrec // _KB
    wid = c * NSUB + s
    slotA = (rowsA, srcbA, dstbA, gsA, ssA)
    slotB = (rowsB, srcbB, dstbB, gsB, ssB)

    def gstart(slot, k):
        rows, srcb, dstb, gsem, _ = slot
        for g in range(CH // 16):
            sl = pl.ds(g * 16, 16)
            srcb[sl] = sarena[k, sl]
            dstb[sl] = darena[k, sl]
        pltpu.async_copy(t1np.at[srcb], rows, gsem)

    def process(slot, k):
        rows, srcb, dstb, gsem, ssem = slot
        pltpu.make_async_copy(t1np.at[srcb], rows, gsem).wait()
        pltpu.async_copy(rows, acc.at[dstb], ssem, add=True)

    def sdone(slot, k):
        rows, srcb, dstb, _, ssem = slot
        pltpu.make_async_copy(rows, acc.at[dstb], ssem).wait()

    def sup(sp, _):
        rec = wid * nrec + sp * _KB
        pltpu.sync_copy(eips.at[pl.ds(rec, _KB)], sarena)
        pltpu.sync_copy(eipd.at[pl.ds(rec, _KB)], darena)
        _pipeline(gstart, process, sdone, slotA, slotB, _KB)
        return _

    lax.fori_loop(0, nsup, sup, 0)
    plsc.subcore_barrier()
    pltpu.sync_copy(acc.at[pl.ds(s * seg, seg)],
                    out.at[pl.ds(c * NP + s * seg, seg)])


_KC = 80


def _sc_edgescalars(eips, eipd, el0, el1, er0, er1, ndt, z1):
    def slot_bufs():
        return [pltpu.VMEM((CH,), _f32) for _ in range(7)] + [
            pltpu.VMEM((CH,), _i32), pltpu.VMEM((CH,), _i32),
            pltpu.SemaphoreType.DMA, pltpu.SemaphoreType.DMA,
            pltpu.SemaphoreType.DMA]

    kfn = pl.kernel(
        _edgescalars_body,
        out_type=(
            jax.ShapeDtypeStruct((HEADS * EP,), _f32),
            jax.ShapeDtypeStruct((4 * NP,), _f32),
            jax.ShapeDtypeStruct((2 * NP,), _f32),
        ),
        mesh=_mesh(),
        scratch_types=[
            pltpu.VMEM_SHARED((NP,), _f32),
            pltpu.VMEM_SHARED((NP,), _f32),
            pltpu.VMEM_SHARED((NP,), _f32),
            pltpu.VMEM((_KC, CH), _i32),
            pltpu.VMEM((_KC, CH), _i32),
        ] + slot_bufs() + slot_bufs(),
    )
    return kfn(eips, eipd, el0, el1, er0, er1, ndt, z1)


def _edgescalars_body(eips, eipd, el0, el1, er0, er1, ndt, z1,
                      ex_o, eap_o, cap_o,
                      eacc0, eacc1, cacc, sarena, darena, *slots):
    c = lax.axis_index("c")
    s = lax.axis_index("s")
    wid = c * NSUB + s
    seg = NP // NSUB
    pltpu.sync_copy(z1.at[pl.ds(s * seg, seg)], eacc0.at[pl.ds(s * seg, seg)])
    pltpu.sync_copy(z1.at[pl.ds(s * seg, seg)], eacc1.at[pl.ds(s * seg, seg)])
    pltpu.sync_copy(z1.at[pl.ds(s * seg, seg)], cacc.at[pl.ds(s * seg, seg)])
    plsc.subcore_barrier()
    slotA = slots[:12]
    slotB = slots[12:]
    tabs = (el0, el1, er0, er1, ndt)

    def bufs_of(slot, k):
        srcb, dstb = slot[7], slot[8]
        idx = [srcb, srcb, dstb, dstb, dstb]
        return list(zip(tabs, idx, slot[:5]))

    def gstart(slot, k):
        srcb, dstb, gsem = slot[7], slot[8], slot[9]
        for g in range(CH // 16):
            sl = pl.ds(g * 16, 16)
            srcb[sl] = sarena[k, sl]
            dstb[sl] = darena[k, sl]
        for tab, idx, buf in bufs_of(slot, k):
            pltpu.async_copy(tab.at[idx], buf, gsem)

    def process(slot, k):
        (el0b, el1b, er0b, er1b, ndb, exb0, exb1, srcb, dstb,
         gsem, ssem, lsem) = slot
        for tab, idx, buf in bufs_of(slot, k):
            pltpu.make_async_copy(tab.at[idx], buf, gsem).wait()
        for g in range(CH // 16):
            sl = pl.ds(g * 16, 16)
            for elb, erb, exb in ((el0b, er0b, exb0), (el1b, er1b, exb1)):
                t = elb[sl] + erb[sl]
                lr = jnp.where(t > 0.0, t, 0.2 * t)
                exb[sl] = jnp.exp(lr)
        base = (wid * _KC + k) * CH
        pltpu.async_copy(exb0, ex_o.at[pl.ds(base, CH)], lsem)
        pltpu.async_copy(exb1, ex_o.at[pl.ds(EP + base, CH)], lsem)
        pltpu.async_copy(exb0, eacc0.at[dstb], ssem, add=True)
        pltpu.async_copy(exb1, eacc1.at[dstb], ssem, add=True)
        pltpu.async_copy(ndb, cacc.at[srcb], ssem, add=True)

    def sdone(slot, k):
        (el0b, el1b, er0b, er1b, ndb, exb0, exb1, srcb, dstb,
         gsem, ssem, lsem) = slot
        base = (wid * _KC + k) * CH
        pltpu.make_async_copy(exb0, ex_o.at[pl.ds(base, CH)], lsem).wait()
        pltpu.make_async_copy(exb1, ex_o.at[pl.ds(EP + base, CH)], lsem).wait()
        pltpu.make_async_copy(exb0, eacc0.at[dstb], ssem).wait()
        pltpu.make_async_copy(exb1, eacc1.at[dstb], ssem).wait()
        pltpu.make_async_copy(ndb, cacc.at[srcb], ssem).wait()

    pltpu.sync_copy(eips.at[pl.ds(wid * _KC, _KC)], sarena)
    pltpu.sync_copy(eipd.at[pl.ds(wid * _KC, _KC)], darena)
    _pipeline(gstart, process, sdone, slotA, slotB, _KC)
    plsc.subcore_barrier()
    pltpu.sync_copy(eacc0.at[pl.ds(s * seg, seg)],
                    eap_o.at[pl.ds((c * 2 + 0) * NP + s * seg, seg)])
    pltpu.sync_copy(eacc1.at[pl.ds(s * seg, seg)],
                    eap_o.at[pl.ds((c * 2 + 1) * NP + s * seg, seg)])
    pltpu.sync_copy(cacc.at[pl.ds(s * seg, seg)],
                    cap_o.at[pl.ds(c * NP + s * seg, seg)])


_KD = 16


def _sc_gat(eips, eipd, ex3, ftp, z2):
    kfn = pl.kernel(
        _gat_body,
        out_type=jax.ShapeDtypeStruct((2 * NP, F), _f32),
        mesh=_mesh(),
        scratch_types=[
            pltpu.VMEM_SHARED((NP, F), _f32),
            pltpu.VMEM((_KD, CH), _i32),
            pltpu.VMEM((_KD, CH), _i32),
            pltpu.VMEM((_KD, CH), _f32),
            pltpu.VMEM((CH, F), _f32),
            pltpu.VMEM((CH,), _i32),
            pltpu.VMEM((CH,), _i32),
            pltpu.VMEM((CH, F), _f32),
            pltpu.VMEM((CH,), _i32),
            pltpu.VMEM((CH,), _i32),
            pltpu.SemaphoreType.DMA,
            pltpu.SemaphoreType.DMA,
            pltpu.SemaphoreType.DMA,
            pltpu.SemaphoreType.DMA,
        ],
    )
    return kfn(eips, eipd, ex3, ftp, z2)


def _gat_body(eips, eipd, ex3, ftp, z2, out, acc, sarena, darena, exarena,
              rowsA, srcbA, dstbA, rowsB, srcbB, dstbB,
              gsA, gsB, ssA, ssB):
    c = lax.axis_index("c")
    s = lax.axis_index("s")
    seg = NP // NSUB
    pltpu.sync_copy(z2.at[pl.ds(s * seg, seg)], acc.at[pl.ds(s * seg, seg)])
    plsc.subcore_barrier()
    nrec = NREC // NSUB
    nsup = nrec // _KD
    delta = c * NP
    slotA = (rowsA, srcbA, dstbA, gsA, ssA)
    slotB = (rowsB, srcbB, dstbB, gsB, ssB)

    def gstart(slot, k):
        rows, srcb, dstb, gsem, _ = slot
        for g in range(CH // 16):
            sl = pl.ds(g * 16, 16)
            srcb[sl] = sarena[k, sl] + delta
            dstb[sl] = darena[k, sl]
        pltpu.async_copy(ftp.at[srcb], rows, gsem)

    def process(slot, k):
        rows, srcb, dstb, gsem, ssem = slot
        pltpu.make_async_copy(ftp.at[srcb], rows, gsem).wait()

        def grp(g, _2):
            sv = exarena[k, pl.ds(g * 16, 16)]
            for e in range(16):
                r = g * 16 + e
                ev = sv[e]
                for j in range(F // 16):
                    sl = pl.ds(j * 16, 16)
                    rows[r, sl] = rows[r, sl] * ev
            return _2

        lax.fori_loop(0, CH // 16, grp, 0)
        pltpu.async_copy(rows, acc.at[dstb], ssem, add=True)

    def sdone(slot, k):
        rows, srcb, dstb, _, ssem = slot
        pltpu.make_async_copy(rows, acc.at[dstb], ssem).wait()

    def sup(sp, _):
        rec = s * nrec + sp * _KD
        pltpu.sync_copy(eips.at[pl.ds(rec, _KD)], sarena)
        pltpu.sync_copy(eipd.at[pl.ds(rec, _KD)], darena)
        exblk = c * (NREC // _KD) + s * nsup + sp
        pltpu.sync_copy(ex3.at[exblk], exarena)
        _pipeline(gstart, process, sdone, slotA, slotB, _KD)
        return _

    lax.fori_loop(0, nsup, sup, 0)
    plsc.subcore_barrier()
    pltpu.sync_copy(acc.at[pl.ds(s * seg, seg)],
                    out.at[pl.ds(c * NP + s * seg, seg)])


_BLK = 1000


def _tc_matmul_xw1(x, w1):
    def body(x_ref, w_ref, o_ref):
        o_ref[...] = jnp.dot(x_ref[...], w_ref[...],
                             preferred_element_type=_f32)

    return pl.pallas_call(
        body,
        grid=(N // _BLK,),
        in_specs=[
            pl.BlockSpec((_BLK, F), lambda i: (i, 0)),
            pl.BlockSpec((F, F), lambda i: (0, 0)),
        ],
        out_specs=pl.BlockSpec((_BLK, F), lambda i: (i, 0)),
        out_shape=jax.ShapeDtypeStruct((N, F), _f32),
    )(x, w1)


def _tc_norms(t1, d00, d01, d10, d11):
    def body(t1_ref, a_ref, b_ref, cc_ref, d_ref, t1n_ref, ns_ref, nd_ref):
        dego = a_ref[:, 0] + b_ref[:, 0]
        degi = cc_ref[:, 0] + d_ref[:, 0]
        ns = lax.rsqrt(jnp.maximum(dego, 1.0))
        nd = lax.rsqrt(jnp.maximum(degi, 1.0))
        t1n_ref[...] = t1_ref[...] * ns[:, None]
        ns_ref[:, 0] = ns
        nd_ref[:, 0] = nd

    vec = pl.BlockSpec((_BLK, 1), lambda i: (i, 0))
    return pl.pallas_call(
        body,
        grid=(N // _BLK,),
        in_specs=[pl.BlockSpec((_BLK, F), lambda i: (i, 0)), vec, vec, vec, vec],
        out_specs=[pl.BlockSpec((_BLK, F), lambda i: (i, 0)), vec, vec],
        out_shape=[
            jax.ShapeDtypeStruct((N, F), _f32),
            jax.ShapeDtypeStruct((N, 1), _f32),
            jax.ShapeDtypeStruct((N, 1), _f32),
        ],
    )(t1, d00, d01, d10, d11)


def _tc_gatprep(m0, m1, nd, b1, wg, al, ar):
    def body(m0_ref, m1_ref, nd_ref, b1_ref, wg_ref, al_ref, ar_ref,
             el_ref, er_ref, f2_ref):
        nd = nd_ref[:, 0]
        m = m0_ref[...] + m1_ref[...]
        h = m * nd[:, None] + b1_ref[0]
        nrm = jnp.sqrt(jnp.sum(h * h, axis=1, keepdims=True))
        h = h / jnp.maximum(nrm, 1e-12)
        h = jax.nn.sigmoid(h)
        feat = jnp.dot(h, wg_ref[...], preferred_element_type=_f32)
        cols = []
        for hh in range(HEADS):
            f = feat[:, hh * F:(hh + 1) * F]
            el = jnp.sum(f * al_ref[hh], axis=1)
            er = jnp.sum(f * ar_ref[hh], axis=1)
            f2_ref[hh] = f
            cols.append((el, er))
        (el0, er0), (el1, er1) = cols
        el_ref[...] = jnp.stack([el0, el1], axis=1)
        er_ref[...] = jnp.stack([er0, er1], axis=1)

    vec = pl.BlockSpec((_BLK, 1), lambda i: (i, 0))
    fullb = pl.BlockSpec((_BLK, F), lambda i: (i, 0))
    return pl.pallas_call(
        body,
        grid=(N // _BLK,),
        in_specs=[
            fullb, fullb, vec,
            pl.BlockSpec((1, F), lambda i: (0, 0)),
            pl.BlockSpec((F, HEADS * F), lambda i: (0, 0)),
            pl.BlockSpec((HEADS, F), lambda i: (0, 0)),
            pl.BlockSpec((HEADS, F), lambda i: (0, 0)),
        ],
        out_specs=[
            pl.BlockSpec((_BLK, 2), lambda i: (i, 0)),
            pl.BlockSpec((_BLK, 2), lambda i: (i, 0)),
            pl.BlockSpec((HEADS, _BLK, F), lambda i: (0, i, 0)),
        ],
        out_shape=[
            jax.ShapeDtypeStruct((N, 2), _f32),
            jax.ShapeDtypeStruct((N, 2), _f32),
            jax.ShapeDtypeStruct((HEADS, N, F), _f32),
        ],
    )(m0, m1, nd, b1, wg, al, ar)


_RBLK = 80


def _tc_readout(num0, num1, e00, e01, e10, e11, cap0, cap1, ns, bg, w2, b2,
                wc, bc):
    nblk = N // _RBLK

    def body(n0_ref, n1_ref, e00_ref, e01_ref, e10_ref, e11_ref,
             c0_ref, c1_ref, ns_ref, bg_ref, w2_ref, b2_ref, wc_ref, bc_ref,
             o_ref, wacc):
        i = pl.program_id(0)

        @pl.when(i == 0)
        def _():
            wacc[...] = jnp.zeros_like(wacc)

        cc = c0_ref[:, 0] + c1_ref[:, 0]
        sw = ns_ref[:, 0] * cc
        ws = []
        for hh, (n_ref, ea, eb) in enumerate(
                ((n0_ref, e00_ref, e10_ref), (n1_ref, e01_ref, e11_ref))):
            esum = ea[:, 0] + eb[:, 0]
            h2 = jax.nn.relu(n_ref[...] / (esum[:, None] + 1e-9) + bg_ref[hh])
            ws.append(jnp.dot(sw[None, :], h2, preferred_element_type=_f32))
        wacc[...] += jnp.concatenate(ws, axis=0)

        @pl.when(i == nblk - 1)
        def _():
            hg = jnp.dot(wacc[...] * (1.0 / N), w2_ref[...],
                         preferred_element_type=_f32) + b2_ref[0]
            o_ref[...] = jnp.dot(hg, wc_ref[...],
                                 preferred_element_type=_f32) + bc_ref[0]

    full = lambda shape: pl.BlockSpec(shape, lambda i: tuple(0 for _ in shape))
    rowb = pl.BlockSpec((_RBLK, F), lambda i: (i, 0))
    colb = pl.BlockSpec((_RBLK, 1), lambda i: (i, 0))
    return pl.pallas_call(
        body,
        grid=(nblk,),
        in_specs=[rowb, rowb, colb, colb, colb, colb, colb, colb, colb,
                  full((HEADS, F)), full((F, F)), full((1, F)),
                  full((F, 16)), full((1, 16))],
        out_specs=full((HEADS, 16)),
        out_shape=jax.ShapeDtypeStruct((HEADS, 16), _f32),
        scratch_shapes=[pltpu.VMEM((HEADS, F), _f32)],
    )(num0, num1, e00, e01, e10, e11, cap0, cap1, ns, bg, w2, b2, wc, bc)


def kernel(x, edge_index, W1, b1, Wg, al, ar, bg, W2, b2, Wc, bc):
    ei = edge_index.reshape(2 * E)
    pad = EP - E
    padidx = (N + (jnp.arange(pad, dtype=_i32) % (NP - N))).astype(_i32)
    srcp = jnp.concatenate([edge_index[0], padidx])
    dstp = jnp.concatenate([edge_index[1], padidx])
    eips = srcp.reshape(NREC, CH)
    eipd = dstp.reshape(NREC, CH)
    z1 = jnp.zeros((NP,), _f32)
    z2 = jnp.zeros((NP, F), _f32)
    ones = jnp.ones((E // NWORK,), _f32)
    zpadrow = jnp.zeros((NP - N, F), _f32)
    zpad = jnp.zeros((NP - N,), _f32)

    deg = _sc_degrees(ei, z1, ones)
    t1 = _tc_matmul_xw1(x, W1)

    dr = lambda k: deg[k * NP:k * NP + N].reshape(N, 1)
    t1n, ns, nd = _tc_norms(t1, dr(0), dr(2), dr(1), dr(3))

    t1np = jnp.concatenate([t1n, zpadrow])
    mflat = _sc_conv1(eips, eipd, t1np, z2)
    m0 = mflat[:N]
    m1 = mflat[NP:NP + N]

    elt, ert, feat2 = _tc_gatprep(m0, m1, nd, b1.reshape(1, F), Wg, al, ar)

    tpad = lambda v: jnp.concatenate([v, zpad])
    exf, eap, cap = _sc_edgescalars(
        eips, eipd, tpad(elt[:, 0]), tpad(elt[:, 1]), tpad(ert[:, 0]),
        tpad(ert[:, 1]), tpad(nd.reshape(N)), z1)

    ftp = jnp.concatenate([feat2[0], zpadrow, feat2[1], zpadrow])
    ex3 = exf.reshape(2 * NREC // _KD, _KD, CH)
    numflat = _sc_gat(eips, eipd, ex3, ftp, z2)

    er = lambda k: eap[k * NP:(k + 1) * NP].reshape(NP, 1)
    out = _tc_readout(
        numflat[:N], numflat[NP:NP + N],
        er(0), er(1), er(2), er(3),
        cap[:NP].reshape(NP, 1), cap[NP:].reshape(NP, 1),
        ns, bg, W2, b2.reshape(1, F), Wc, bc.reshape(1, 16))
    return out.reshape(1, HEADS, 16)
